# baseline (speedup 1.0000x reference)
import functools

import jax
import jax.numpy as jnp
from jax.experimental import pallas as pl
from jax.experimental.pallas import tpu as pltpu
from jax.experimental.pallas import tpu_sc as plsc

D = 512
H = 256
_HW = 128
_INTERPRET = False


def _ln(x, g, b, eps=1e-5):
    m = x.mean(-1, keepdims=True)
    v = ((x - m) ** 2).mean(-1, keepdims=True)
    return (x - m) / jnp.sqrt(v + eps) * g + b


def _gelu(x):
    return x * 0.5 * (1.0 + jax.lax.erf(x * (2.0 ** -0.5)))


def _full2d(a):
    return pl.BlockSpec(a.shape, lambda i: (0, 0))


def _fuse_proj_body(xg_ref, gt_ref, tm_ref, gate_ref, wfg_ref, wft_ref, bf_ref,
                    w1_ref, b1_ref, g1_ref, n1_ref, w2_ref, b2_ref, g2_ref,
                    n2_ref, out_ref):
    xg = xg_ref[...]
    fused = _gelu(xg @ wfg_ref[...] + gt_ref[...] @ wft_ref[...] + bf_ref[...])
    g = xg + gate_ref[0, 0] * (tm_ref[...] * fused)
    h = g @ w1_ref[...] + b1_ref[...]
    h = _gelu(_ln(h, g1_ref[...], n1_ref[...]))
    h = h @ w2_ref[...] + b2_ref[...]
    h = _ln(h, g2_ref[...], n2_ref[...])
    out_ref[...] = _aux_halves(h)


def _aux_halves(h):
    return jnp.stack([h[:, :128], h[:, 128:]])


def _fuse_proj(xg, gt, tm, gate, fuse_p, proj_p, bm):
    n = xg.shape[0]
    wfg = fuse_p['W'][:D]
    wft = fuse_p['W'][D:]
    args = (xg, gt, tm.reshape(n, 1), gate.reshape(1, 1), wfg, wft,
            fuse_p['b'].reshape(1, D),
            proj_p['W1'], proj_p['b1'].reshape(1, H),
            proj_p['g1'].reshape(1, H), proj_p['bn1'].reshape(1, H),
            proj_p['W2'], proj_p['b2'].reshape(1, H),
            proj_p['g2'].reshape(1, H), proj_p['bn2'].reshape(1, H))
    in_specs = [
        pl.BlockSpec((bm, D), lambda i: (i, 0)),
        pl.BlockSpec((bm, D), lambda i: (i, 0)),
        pl.BlockSpec((bm, 1), lambda i: (i, 0)),
        pl.BlockSpec((1, 1), lambda i: (0, 0)),
    ] + [_full2d(a) for a in args[4:]]
    return pl.pallas_call(
        _fuse_proj_body,
        grid=(n // bm,),
        in_specs=in_specs,
        out_specs=pl.BlockSpec((2, bm, _HW), lambda i: (0, i, 0)),
        out_shape=jax.ShapeDtypeStruct((2, n, _HW), jnp.float32),
        interpret=_INTERPRET,
    )(*args)


def _proj_body(x_ref, w1_ref, b1_ref, g1_ref, n1_ref, w2_ref, b2_ref, g2_ref,
               n2_ref, out_ref):
    h = x_ref[...] @ w1_ref[...] + b1_ref[...]
    h = _gelu(_ln(h, g1_ref[...], n1_ref[...]))
    h = h @ w2_ref[...] + b2_ref[...]
    h = _ln(h, g2_ref[...], n2_ref[...])
    out_ref[...] = _aux_halves(h)


def _proj(x, p, bm):
    n = x.shape[0]
    args = (x, p['W1'], p['b1'].reshape(1, H), p['g1'].reshape(1, H),
            p['bn1'].reshape(1, H), p['W2'], p['b2'].reshape(1, H),
            p['g2'].reshape(1, H), p['bn2'].reshape(1, H))
    in_specs = [pl.BlockSpec((bm, D), lambda i: (i, 0))] + \
               [_full2d(a) for a in args[1:]]
    return pl.pallas_call(
        _proj_body,
        grid=(n // bm,),
        in_specs=in_specs,
        out_specs=pl.BlockSpec((2, bm, _HW), lambda i: (0, i, 0)),
        out_shape=jax.ShapeDtypeStruct((2, n, _HW), jnp.float32),
        interpret=_INTERPRET,
    )(*args)


def _combine_body(k, halves_out, *refs):
    h_ref = refs[0]
    s_refs = refs[1:1 + k]
    c_refs = refs[1 + k:1 + 2 * k]
    wl_refs = refs[1 + 2 * k:1 + 3 * k]
    wr_refs = refs[1 + 3 * k:1 + 4 * k]
    bl_ref, g_ref, b_ref = refs[1 + 4 * k:1 + 4 * k + 3]
    out_ref = refs[-1]
    h = jnp.concatenate([h_ref[0], h_ref[1]], axis=-1)
    o = jnp.zeros_like(h)
    for s_ref, c_ref, wl_ref in zip(s_refs, c_refs, wl_refs):
        s = jnp.concatenate([s_ref[0], s_ref[1]], axis=-1)
        cnt = c_ref[0, :, 0:1] + c_ref[1, :, 0:1]
        mean = s / jnp.maximum(cnt, 1.0)
        o = o + mean @ wl_ref[...]
    wr = wr_refs[0][...]
    for r in wr_refs[1:]:
        wr = wr + r[...]
    o = o + h @ wr + bl_ref[...]
    res = _ln(_gelu(o) + h, g_ref[...], b_ref[...])
    if halves_out:
        out_ref[...] = _aux_halves(res)
    else:
        out_ref[...] = res


def _combine(h2, sums, cnts, wls, wrs, bls, g, b, bm, halves_out):
    k = len(sums)
    n = h2.shape[1]
    bl = bls[0]
    for x in bls[1:]:
        bl = bl + x
    args = ([h2] + list(sums) + list(cnts) + list(wls) + list(wrs)
            + [bl.reshape(1, H), g.reshape(1, H), b.reshape(1, H)])
    in_specs = ([pl.BlockSpec((2, bm, _HW), lambda i: (0, i, 0))]
                + [pl.BlockSpec((2, bm, _HW), lambda i: (0, i, 0))] * k
                + [pl.BlockSpec((2, bm, _HW), lambda i: (0, i, 0))] * k
                + [_full2d(a) for a in args[1 + 2 * k:]])
    if halves_out:
        out_spec = pl.BlockSpec((2, bm, _HW), lambda i: (0, i, 0))
        out_shape = jax.ShapeDtypeStruct((2, n, _HW), jnp.float32)
    else:
        out_spec = pl.BlockSpec((bm, H), lambda i: (i, 0))
        out_shape = jax.ShapeDtypeStruct((n, H), jnp.float32)
    return pl.pallas_call(
        functools.partial(_combine_body, k, halves_out),
        grid=(n // bm,),
        in_specs=in_specs,
        out_specs=out_spec,
        out_shape=out_shape,
        interpret=_INTERPRET,
    )(*args)


def _head_body(hc_ref, ctx_ref, w1c_ref, w1x_ref, b1_ref, w2_ref, b2_ref,
               out_ref):
    z = _gelu(hc_ref[...] @ w1c_ref[...] + ctx_ref[...] @ w1x_ref[...]
              + b1_ref[...])
    out_ref[...] = z @ w2_ref[...] + b2_ref[0, 0]


def _head(hc, ctx, p, bm):
    n = hc.shape[0]
    w2p = jnp.pad(p['W2'], ((0, 0), (0, 127)))
    args = (hc, ctx, p['W1'][:H], p['W1'][H:], p['b1'].reshape(1, H), w2p,
            p['b2'].reshape(1, 1))
    in_specs = [pl.BlockSpec((bm, H), lambda i: (i, 0)),
                pl.BlockSpec((bm, H), lambda i: (i, 0))] + \
               [_full2d(a) for a in args[2:]]
    out = pl.pallas_call(
        _head_body,
        grid=(n // bm,),
        in_specs=in_specs,
        out_specs=pl.BlockSpec((bm, 128), lambda i: (i, 0)),
        out_shape=jax.ShapeDtypeStruct((n, 128), jnp.float32),
        interpret=_INTERPRET,
    )(*args)
    return out[:, 0]


_CHUNK = 128
_WCH = 64
_NBUF = 4


def _sc_block_sums(h2f, ets, srcs, dsts, ones_in, dst_n, with_counts):
    mesh = plsc.VectorSubcoreMesh(core_axis_name="c", subcore_axis_name="s")
    nt = len(ets)
    max_n = max(dst_n[et.split('2')[1]] for et in ets)
    out_type = [jax.ShapeDtypeStruct((2, dst_n[et.split('2')[1]], _HW),
                                     jnp.float32) for et in ets]
    if with_counts:
        out_type = out_type + [
            jax.ShapeDtypeStruct((2, dst_n[et.split('2')[1]], _HW),
                                 jnp.float32) for et in ets]
    tbls = [h2f['g'], h2f['l'], h2f['c']]
    tbl_of = {'g': 0, 'l': 1, 'c': 2}

    @functools.partial(
        pl.kernel, mesh=mesh, out_type=out_type,
        scratch_types=[
            pltpu.VMEM((_CHUNK,), jnp.int32),
            pltpu.VMEM((_CHUNK,), jnp.int32),
            pltpu.VMEM((_CHUNK,), jnp.int32),
            pltpu.VMEM((_CHUNK,), jnp.int32),
            pltpu.VMEM((1, _CHUNK), jnp.int32),
            pltpu.VMEM((1, _CHUNK), jnp.int32),
            pltpu.VMEM((_CHUNK, _HW), jnp.float32),
            pltpu.VMEM((_CHUNK, _HW), jnp.float32),
            pltpu.VMEM((_WCH, _HW), jnp.float32),
            pltpu.VMEM((_CHUNK, _HW), jnp.float32),
            pltpu.VMEM_SHARED((max_n, _HW), jnp.float32),
            pltpu.SemaphoreType.DMA,
            pltpu.SemaphoreType.DMA,
            pltpu.SemaphoreType.DMA,
            pltpu.SemaphoreType.DMA,
        ],
    )
    def k(*refs):
        tbl_refs = refs[:3]
        src_refs = refs[3:3 + nt]
        dst_refs = refs[3 + nt:3 + 2 * nt]
        ones_hbm = refs[3 + 2 * nt]
        out_refs = refs[4 + 2 * nt:4 + 3 * nt]
        no = 4 + 3 * nt + (nt if with_counts else 0)
        cnt_refs = refs[4 + 3 * nt:no]
        (srcb0, srcb1, srca0, srca1, dstb0, dstb1, rowsb0, rowsb1,
         zb, ones, acc, gsem0, gsem1, ssem0, ssem1) = refs[no:]
        c = jax.lax.axis_index("c")
        s = jax.lax.axis_index("s")
        w = c * 16 + s
        pltpu.sync_copy(ones_hbm, ones)

        def fill_z(i, _):
            for j in range(_HW // 16):
                zb[i, pl.ds(j * 16, 16)] = jnp.zeros((16,), jnp.float32)
            return 0
        jax.lax.fori_loop(0, _WCH, fill_z, 0)

        for t, et in enumerate(ets):
            skey, dkey = et.split('2')
            tref = tbl_refs[tbl_of[skey]]
            sref, dref, oref = src_refs[t], dst_refs[t], out_refs[t]
            nsrc = tref.shape[0] // 2
            e = sref.shape[0]
            n = oref.shape[1]
            rows = n // 16
            et_per_tile = e // 16
            for j in range(rows // _WCH):
                pltpu.sync_copy(zb, acc.at[pl.ds(s * rows + j * _WCH, _WCH)])
            plsc.subcore_barrier()

            base = s * et_per_tile
            nch = et_per_tile // _CHUNK

            def prep(off, sb, sa, db, rb, sem):
                pltpu.sync_copy(sref.at[pl.ds(off, _CHUNK)], sb)
                pltpu.sync_copy(dref.at[pl.ds(off, _CHUNK)], db.at[0])
                for j in range(_CHUNK // 16):
                    sa[pl.ds(j * 16, 16)] = sb[pl.ds(j * 16, 16)] + c * nsrc
                pltpu.async_copy(tref.at[sa], rb, sem)

            def fin(sa, db, rb, gsem, ssem):
                pltpu.make_async_copy(tref.at[sa], rb, gsem).wait()
                pltpu.async_copy(rb, acc.at[db.at[0]], ssem, add=True)

            def drain(db, rb, ssem):
                pltpu.make_async_copy(rb, acc.at[db.at[0]], ssem).wait()

            prep(base, srcb0, srca0, dstb0, rowsb0, gsem0)

            def body(i2, _):
                @pl.when(i2 > 0)
                def _drainb():
                    drain(dstb1, rowsb1, ssem1)
                prep(base + (2 * i2 + 1) * _CHUNK,
                     srcb1, srca1, dstb1, rowsb1, gsem1)
                fin(srca0, dstb0, rowsb0, gsem0, ssem0)

                @pl.when(i2 + 1 < nch // 2)
                def _nexta():
                    drain(dstb0, rowsb0, ssem0)
                    prep(base + (2 * i2 + 2) * _CHUNK,
                         srcb0, srca0, dstb0, rowsb0, gsem0)
                fin(srca1, dstb1, rowsb1, gsem1, ssem1)
                return 0
            jax.lax.fori_loop(0, nch // 2, body, 0)
            drain(dstb0, rowsb0, ssem0)
            drain(dstb1, rowsb1, ssem1)
            plsc.subcore_barrier()
            for j in range(rows // _WCH):
                r0 = s * rows + j * _WCH
                pltpu.sync_copy(acc.at[pl.ds(r0, _WCH)],
                                oref.at[c, pl.ds(r0, _WCH)])
            plsc.subcore_barrier()

            if not with_counts:
                continue
            cref = cnt_refs[t]
            for j in range(rows // _WCH):
                pltpu.sync_copy(zb, acc.at[pl.ds(s * rows + j * _WCH, _WCH)])
            plsc.subcore_barrier()
            ew = e // 32

            def cbody(i, _):
                off = w * ew + i * _CHUNK
                pltpu.sync_copy(dref.at[pl.ds(off, _CHUNK)], dstb0.at[0])
                pltpu.sync_copy(ones, acc.at[dstb0.at[0]], add=True)
                return 0
            jax.lax.fori_loop(0, ew // _CHUNK, cbody, 0)
            plsc.subcore_barrier()
            for j in range(rows // _WCH):
                r0 = s * rows + j * _WCH
                pltpu.sync_copy(acc.at[pl.ds(r0, _WCH)],
                                cref.at[c, pl.ds(r0, _WCH)])
            plsc.subcore_barrier()

    return k(*tbls, *srcs, *dsts, ones_in)


def _sc_row_gather(tbl, idx):
    q = idx.shape[0]
    mesh = plsc.VectorSubcoreMesh(core_axis_name="c", subcore_axis_name="s")
    qw = q // 32

    @functools.partial(
        pl.kernel, mesh=mesh,
        out_type=jax.ShapeDtypeStruct((q, 256), jnp.float32),
        scratch_types=[
            pltpu.VMEM((_CHUNK,), jnp.int32),
            pltpu.VMEM((_CHUNK, 256), jnp.float32),
            pltpu.SemaphoreType.DMA,
        ],
    )
    def k(tbl_ref, idx_ref, out_ref, idxb, rowsb, sem):
        c = jax.lax.axis_index("c")
        s = jax.lax.axis_index("s")
        w = c * 16 + s

        def body(i, _):
            off = w * qw + i * _CHUNK
            pltpu.sync_copy(idx_ref.at[pl.ds(off, _CHUNK)], idxb)
            pltpu.async_copy(tbl_ref.at[idxb], rowsb, sem).wait()
            pltpu.sync_copy(rowsb, out_ref.at[pl.ds(off, _CHUNK)])
            return 0
        jax.lax.fori_loop(0, qw // _CHUNK, body, 0)

    return k(tbl, idx)


def kernel(x_global, x_lesion, x_cause, global_txt, text_mask, ei_g2l, ei_l2g,
           ei_l2l, ei_l2c, ei_c2l, ei_g2c, ei_c2g, cause_batch, params):
    B = x_global.shape[0]
    NL = x_lesion.shape[0]
    NC = x_cause.shape[0]
    eis = {'g2l': ei_g2l, 'l2g': ei_l2g, 'l2l': ei_l2l, 'l2c': ei_l2c,
           'c2l': ei_c2l, 'g2c': ei_g2c, 'c2g': ei_c2g}
    eis = {k: v.astype(jnp.int32) for k, v in eis.items()}
    dst_n = {'g': B, 'l': NL, 'c': NC}

    et_order = list(eis.keys())

    hg2 = _fuse_proj(x_global, global_txt, text_mask, params['gate'],
                     params['fuse'], params['proj_global'], bm=256)
    hl2 = _proj(x_lesion, params['proj_lesion'], bm=512)
    hc2 = _proj(x_cause, params['proj_cause'], bm=512)

    h2 = {'g': hg2, 'l': hl2, 'c': hc2}
    ones_in = jnp.ones((_CHUNK, _HW), jnp.float32)
    for bi, bp in enumerate(params['blocks']):
        h2f = {k: v.reshape(2 * v.shape[1], _HW) for k, v in h2.items()}
        out_list = _sc_block_sums(h2f, et_order,
                                  [eis[e][0] for e in et_order],
                                  [eis[e][1] for e in et_order],
                                  ones_in, dst_n, with_counts=(bi == 0))
        sums = dict(zip(et_order, out_list[:len(et_order)]))
        if bi == 0:
            cnts = dict(zip(et_order, out_list[len(et_order):]))
        last = bi == len(params['blocks']) - 1
        new = {}
        for dst, ets, bmv in (('l', ('g2l', 'l2l', 'c2l'), 512),
                              ('g', ('l2g', 'c2g'), 256),
                              ('c', ('l2c', 'g2c'), 512)):
            new[dst] = _combine(
                h2[dst], [sums[e] for e in ets], [cnts[e] for e in ets],
                [bp[e]['Wl'] for e in ets], [bp[e]['Wr'] for e in ets],
                [bp[e]['bl'] for e in ets],
                bp['n' + dst + '_g'], bp['n' + dst + '_b'],
                bm=bmv, halves_out=not last)
        h2 = new

    hg, hl, hc = h2['g'], h2['l'], h2['c']
    ctx = _sc_row_gather(hg, cause_batch.astype(jnp.int32))
    scores = _head(hc, ctx, params['head'], bm=512)
    return scores, hc, hg, hl


# speedup vs baseline: 1.0004x; 1.0004x over previous
import functools

import jax
import jax.numpy as jnp
from jax.experimental import pallas as pl
from jax.experimental.pallas import tpu as pltpu
from jax.experimental.pallas import tpu_sc as plsc

D = 512
H = 256
_HW = 128
_INTERPRET = False


def _ln(x, g, b, eps=1e-5):
    m = x.mean(-1, keepdims=True)
    v = ((x - m) ** 2).mean(-1, keepdims=True)
    return (x - m) / jnp.sqrt(v + eps) * g + b


def _gelu(x):
    return x * 0.5 * (1.0 + jax.lax.erf(x * (2.0 ** -0.5)))


def _full2d(a):
    return pl.BlockSpec(a.shape, lambda i: (0, 0))


def _fuse_proj_body(xg_ref, gt_ref, tm_ref, gate_ref, wfg_ref, wft_ref, bf_ref,
                    w1_ref, b1_ref, g1_ref, n1_ref, w2_ref, b2_ref, g2_ref,
                    n2_ref, out_ref):
    xg = xg_ref[...]
    fused = _gelu(xg @ wfg_ref[...] + gt_ref[...] @ wft_ref[...] + bf_ref[...])
    g = xg + gate_ref[0, 0] * (tm_ref[...] * fused)
    h = g @ w1_ref[...] + b1_ref[...]
    h = _gelu(_ln(h, g1_ref[...], n1_ref[...]))
    h = h @ w2_ref[...] + b2_ref[...]
    h = _ln(h, g2_ref[...], n2_ref[...])
    out_ref[...] = _aux_halves(h)


def _aux_halves(h):
    return jnp.stack([h[:, :128], h[:, 128:]])


def _fuse_proj(xg, gt, tm, gate, fuse_p, proj_p, bm):
    n = xg.shape[0]
    wfg = fuse_p['W'][:D]
    wft = fuse_p['W'][D:]
    args = (xg, gt, tm.reshape(n, 1), gate.reshape(1, 1), wfg, wft,
            fuse_p['b'].reshape(1, D),
            proj_p['W1'], proj_p['b1'].reshape(1, H),
            proj_p['g1'].reshape(1, H), proj_p['bn1'].reshape(1, H),
            proj_p['W2'], proj_p['b2'].reshape(1, H),
            proj_p['g2'].reshape(1, H), proj_p['bn2'].reshape(1, H))
    in_specs = [
        pl.BlockSpec((bm, D), lambda i: (i, 0)),
        pl.BlockSpec((bm, D), lambda i: (i, 0)),
        pl.BlockSpec((bm, 1), lambda i: (i, 0)),
        pl.BlockSpec((1, 1), lambda i: (0, 0)),
    ] + [_full2d(a) for a in args[4:]]
    return pl.pallas_call(
        _fuse_proj_body,
        grid=(n // bm,),
        in_specs=in_specs,
        out_specs=pl.BlockSpec((2, bm, _HW), lambda i: (0, i, 0)),
        out_shape=jax.ShapeDtypeStruct((2, n, _HW), jnp.float32),
        interpret=_INTERPRET,
    )(*args)


def _proj_body(x_ref, w1_ref, b1_ref, g1_ref, n1_ref, w2_ref, b2_ref, g2_ref,
               n2_ref, out_ref):
    h = x_ref[...] @ w1_ref[...] + b1_ref[...]
    h = _gelu(_ln(h, g1_ref[...], n1_ref[...]))
    h = h @ w2_ref[...] + b2_ref[...]
    h = _ln(h, g2_ref[...], n2_ref[...])
    out_ref[...] = _aux_halves(h)


def _proj(x, p, bm):
    n = x.shape[0]
    args = (x, p['W1'], p['b1'].reshape(1, H), p['g1'].reshape(1, H),
            p['bn1'].reshape(1, H), p['W2'], p['b2'].reshape(1, H),
            p['g2'].reshape(1, H), p['bn2'].reshape(1, H))
    in_specs = [pl.BlockSpec((bm, D), lambda i: (i, 0))] + \
               [_full2d(a) for a in args[1:]]
    return pl.pallas_call(
        _proj_body,
        grid=(n // bm,),
        in_specs=in_specs,
        out_specs=pl.BlockSpec((2, bm, _HW), lambda i: (0, i, 0)),
        out_shape=jax.ShapeDtypeStruct((2, n, _HW), jnp.float32),
        interpret=_INTERPRET,
    )(*args)


def _combine_body(k, halves_out, *refs):
    h_ref = refs[0]
    s_refs = refs[1:1 + k]
    c_refs = refs[1 + k:1 + 2 * k]
    wl_refs = refs[1 + 2 * k:1 + 3 * k]
    wr_refs = refs[1 + 3 * k:1 + 4 * k]
    bl_ref, g_ref, b_ref = refs[1 + 4 * k:1 + 4 * k + 3]
    out_ref = refs[-1]
    h = jnp.concatenate([h_ref[0], h_ref[1]], axis=-1)
    o = jnp.zeros_like(h)
    for s_ref, c_ref, wl_ref in zip(s_refs, c_refs, wl_refs):
        s = jnp.concatenate([s_ref[0], s_ref[1]], axis=-1)
        cnt = c_ref[0, :, 0:1] + c_ref[1, :, 0:1]
        mean = s / jnp.maximum(cnt, 1.0)
        o = o + mean @ wl_ref[...]
    wr = wr_refs[0][...]
    for r in wr_refs[1:]:
        wr = wr + r[...]
    o = o + h @ wr + bl_ref[...]
    res = _ln(_gelu(o) + h, g_ref[...], b_ref[...])
    if halves_out:
        out_ref[...] = _aux_halves(res)
    else:
        out_ref[...] = res


def _combine(h2, sums, cnts, wls, wrs, bls, g, b, bm, halves_out):
    k = len(sums)
    n = h2.shape[1]
    bl = bls[0]
    for x in bls[1:]:
        bl = bl + x
    args = ([h2] + list(sums) + list(cnts) + list(wls) + list(wrs)
            + [bl.reshape(1, H), g.reshape(1, H), b.reshape(1, H)])
    in_specs = ([pl.BlockSpec((2, bm, _HW), lambda i: (0, i, 0))]
                + [pl.BlockSpec((2, bm, _HW), lambda i: (0, i, 0))] * k
                + [pl.BlockSpec((2, bm, _HW), lambda i: (0, i, 0))] * k
                + [_full2d(a) for a in args[1 + 2 * k:]])
    if halves_out:
        out_spec = pl.BlockSpec((2, bm, _HW), lambda i: (0, i, 0))
        out_shape = jax.ShapeDtypeStruct((2, n, _HW), jnp.float32)
    else:
        out_spec = pl.BlockSpec((bm, H), lambda i: (i, 0))
        out_shape = jax.ShapeDtypeStruct((n, H), jnp.float32)
    return pl.pallas_call(
        functools.partial(_combine_body, k, halves_out),
        grid=(n // bm,),
        in_specs=in_specs,
        out_specs=out_spec,
        out_shape=out_shape,
        interpret=_INTERPRET,
    )(*args)


def _head_body(hc_ref, ctx_ref, w1c_ref, w1x_ref, b1_ref, w2_ref, b2_ref,
               out_ref):
    z = _gelu(hc_ref[...] @ w1c_ref[...] + ctx_ref[...] @ w1x_ref[...]
              + b1_ref[...])
    out_ref[...] = z @ w2_ref[...] + b2_ref[0, 0]


def _head(hc, ctx, p, bm):
    n = hc.shape[0]
    w2p = jnp.pad(p['W2'], ((0, 0), (0, 127)))
    args = (hc, ctx, p['W1'][:H], p['W1'][H:], p['b1'].reshape(1, H), w2p,
            p['b2'].reshape(1, 1))
    in_specs = [pl.BlockSpec((bm, H), lambda i: (i, 0)),
                pl.BlockSpec((bm, H), lambda i: (i, 0))] + \
               [_full2d(a) for a in args[2:]]
    out = pl.pallas_call(
        _head_body,
        grid=(n // bm,),
        in_specs=in_specs,
        out_specs=pl.BlockSpec((bm, 128), lambda i: (i, 0)),
        out_shape=jax.ShapeDtypeStruct((n, 128), jnp.float32),
        interpret=_INTERPRET,
    )(*args)
    return out[:, 0]


_CHUNK = 128
_WCH = 64
_NBUF = 4


def _sc_block_sums(h2f, ets, srcs, dsts, ones_in, dst_n, with_counts):
    mesh = plsc.VectorSubcoreMesh(core_axis_name="c", subcore_axis_name="s")
    nt = len(ets)
    max_n = max(dst_n[et.split('2')[1]] for et in ets)
    out_type = [jax.ShapeDtypeStruct((2, dst_n[et.split('2')[1]], _HW),
                                     jnp.float32) for et in ets]
    if with_counts:
        out_type = out_type + [
            jax.ShapeDtypeStruct((2, dst_n[et.split('2')[1]], _HW),
                                 jnp.float32) for et in ets]
    tbls = [h2f['g'], h2f['l'], h2f['c']]
    tbl_of = {'g': 0, 'l': 1, 'c': 2}

    @functools.partial(
        pl.kernel, mesh=mesh, out_type=out_type,
        scratch_types=[
            pltpu.VMEM((_CHUNK,), jnp.int32),
            pltpu.VMEM((_CHUNK,), jnp.int32),
            pltpu.VMEM((_CHUNK,), jnp.int32),
            pltpu.VMEM((_CHUNK,), jnp.int32),
            pltpu.VMEM((1, _CHUNK), jnp.int32),
            pltpu.VMEM((1, _CHUNK), jnp.int32),
            pltpu.VMEM((_CHUNK, _HW), jnp.float32),
            pltpu.VMEM((_CHUNK, _HW), jnp.float32),
            pltpu.VMEM((_WCH, _HW), jnp.float32),
            pltpu.VMEM((_CHUNK, _HW), jnp.float32),
            pltpu.VMEM_SHARED((max_n, _HW), jnp.float32),
            pltpu.SemaphoreType.DMA,
            pltpu.SemaphoreType.DMA,
            pltpu.SemaphoreType.DMA,
            pltpu.SemaphoreType.DMA,
        ],
    )
    def k(*refs):
        tbl_refs = refs[:3]
        src_refs = refs[3:3 + nt]
        dst_refs = refs[3 + nt:3 + 2 * nt]
        ones_hbm = refs[3 + 2 * nt]
        out_refs = refs[4 + 2 * nt:4 + 3 * nt]
        no = 4 + 3 * nt + (nt if with_counts else 0)
        cnt_refs = refs[4 + 3 * nt:no]
        (srcb0, srcb1, srca0, srca1, dstb0, dstb1, rowsb0, rowsb1,
         zb, ones, acc, gsem0, gsem1, ssem0, ssem1) = refs[no:]
        c = jax.lax.axis_index("c")
        s = jax.lax.axis_index("s")
        w = c * 16 + s
        pltpu.sync_copy(ones_hbm, ones)

        def fill_z(i, _):
            for j in range(_HW // 16):
                zb[i, pl.ds(j * 16, 16)] = jnp.zeros((16,), jnp.float32)
            return 0
        jax.lax.fori_loop(0, _WCH, fill_z, 0)

        for t, et in enumerate(ets):
            skey, dkey = et.split('2')
            tref = tbl_refs[tbl_of[skey]]
            sref, dref, oref = src_refs[t], dst_refs[t], out_refs[t]
            nsrc = tref.shape[0] // 2
            e = sref.shape[0]
            n = oref.shape[1]
            rows = n // 16
            et_per_tile = e // 16
            for j in range(rows // _WCH):
                pltpu.sync_copy(zb, acc.at[pl.ds(s * rows + j * _WCH, _WCH)])
            plsc.subcore_barrier()

            base = s * et_per_tile
            nch = et_per_tile // _CHUNK

            def prep(off, sb, sa, db, rb, sem):
                pltpu.sync_copy(sref.at[pl.ds(off, _CHUNK)], sb)
                pltpu.sync_copy(dref.at[pl.ds(off, _CHUNK)], db.at[0])
                for j in range(_CHUNK // 16):
                    sa[pl.ds(j * 16, 16)] = sb[pl.ds(j * 16, 16)] + c * nsrc
                pltpu.async_copy(tref.at[sa], rb, sem)

            def fin(sa, db, rb, gsem, ssem):
                pltpu.make_async_copy(tref.at[sa], rb, gsem).wait()
                pltpu.async_copy(rb, acc.at[db.at[0]], ssem, add=True)

            def drain(db, rb, ssem):
                pltpu.make_async_copy(rb, acc.at[db.at[0]], ssem).wait()

            prep(base, srcb0, srca0, dstb0, rowsb0, gsem0)

            def body(i2, _):
                @pl.when(i2 > 0)
                def _drainb():
                    drain(dstb1, rowsb1, ssem1)
                prep(base + (2 * i2 + 1) * _CHUNK,
                     srcb1, srca1, dstb1, rowsb1, gsem1)
                fin(srca0, dstb0, rowsb0, gsem0, ssem0)

                @pl.when(i2 + 1 < nch // 2)
                def _nexta():
                    drain(dstb0, rowsb0, ssem0)
                    prep(base + (2 * i2 + 2) * _CHUNK,
                         srcb0, srca0, dstb0, rowsb0, gsem0)
                fin(srca1, dstb1, rowsb1, gsem1, ssem1)
                return 0
            jax.lax.fori_loop(0, nch // 2, body, 0)
            drain(dstb0, rowsb0, ssem0)
            drain(dstb1, rowsb1, ssem1)
            plsc.subcore_barrier()
            for j in range(rows // _WCH):
                r0 = s * rows + j * _WCH
                pltpu.sync_copy(acc.at[pl.ds(r0, _WCH)],
                                oref.at[c, pl.ds(r0, _WCH)])
            plsc.subcore_barrier()

            if not with_counts:
                continue
            cref = cnt_refs[t]
            for j in range(rows // _WCH):
                pltpu.sync_copy(zb, acc.at[pl.ds(s * rows + j * _WCH, _WCH)])
            plsc.subcore_barrier()
            ew = e // 32

            ncch = ew // _CHUNK

            def cdrain(db, ssem):
                pltpu.make_async_copy(ones, acc.at[db.at[0]], ssem).wait()

            pltpu.sync_copy(dref.at[pl.ds(w * ew, _CHUNK)], dstb0.at[0])

            def cbody(i2, _):
                @pl.when(i2 > 0)
                def _cd1():
                    cdrain(dstb1, ssem1)
                pltpu.sync_copy(
                    dref.at[pl.ds(w * ew + (2 * i2 + 1) * _CHUNK, _CHUNK)],
                    dstb1.at[0])
                pltpu.async_copy(ones, acc.at[dstb0.at[0]], ssem0, add=True)
                pltpu.async_copy(ones, acc.at[dstb1.at[0]], ssem1, add=True)

                @pl.when(i2 + 1 < ncch // 2)
                def _cnext():
                    cdrain(dstb0, ssem0)
                    pltpu.sync_copy(
                        dref.at[pl.ds(w * ew + (2 * i2 + 2) * _CHUNK,
                                      _CHUNK)], dstb0.at[0])
                return 0
            jax.lax.fori_loop(0, ncch // 2, cbody, 0)
            cdrain(dstb0, ssem0)
            cdrain(dstb1, ssem1)
            plsc.subcore_barrier()
            for j in range(rows // _WCH):
                r0 = s * rows + j * _WCH
                pltpu.sync_copy(acc.at[pl.ds(r0, _WCH)],
                                cref.at[c, pl.ds(r0, _WCH)])
            plsc.subcore_barrier()

    return k(*tbls, *srcs, *dsts, ones_in)


def _sc_row_gather(tbl, idx):
    q = idx.shape[0]
    mesh = plsc.VectorSubcoreMesh(core_axis_name="c", subcore_axis_name="s")
    qw = q // 32

    @functools.partial(
        pl.kernel, mesh=mesh,
        out_type=jax.ShapeDtypeStruct((q, 256), jnp.float32),
        scratch_types=[
            pltpu.VMEM((_CHUNK,), jnp.int32),
            pltpu.VMEM((_CHUNK, 256), jnp.float32),
            pltpu.SemaphoreType.DMA,
        ],
    )
    def k(tbl_ref, idx_ref, out_ref, idxb, rowsb, sem):
        c = jax.lax.axis_index("c")
        s = jax.lax.axis_index("s")
        w = c * 16 + s

        def body(i, _):
            off = w * qw + i * _CHUNK
            pltpu.sync_copy(idx_ref.at[pl.ds(off, _CHUNK)], idxb)
            pltpu.async_copy(tbl_ref.at[idxb], rowsb, sem).wait()
            pltpu.sync_copy(rowsb, out_ref.at[pl.ds(off, _CHUNK)])
            return 0
        jax.lax.fori_loop(0, qw // _CHUNK, body, 0)

    return k(tbl, idx)


def kernel(x_global, x_lesion, x_cause, global_txt, text_mask, ei_g2l, ei_l2g,
           ei_l2l, ei_l2c, ei_c2l, ei_g2c, ei_c2g, cause_batch, params):
    B = x_global.shape[0]
    NL = x_lesion.shape[0]
    NC = x_cause.shape[0]
    eis = {'g2l': ei_g2l, 'l2g': ei_l2g, 'l2l': ei_l2l, 'l2c': ei_l2c,
           'c2l': ei_c2l, 'g2c': ei_g2c, 'c2g': ei_c2g}
    eis = {k: v.astype(jnp.int32) for k, v in eis.items()}
    dst_n = {'g': B, 'l': NL, 'c': NC}

    et_order = list(eis.keys())

    hg2 = _fuse_proj(x_global, global_txt, text_mask, params['gate'],
                     params['fuse'], params['proj_global'], bm=256)
    hl2 = _proj(x_lesion, params['proj_lesion'], bm=512)
    hc2 = _proj(x_cause, params['proj_cause'], bm=512)

    h2 = {'g': hg2, 'l': hl2, 'c': hc2}
    ones_in = jnp.ones((_CHUNK, _HW), jnp.float32)
    for bi, bp in enumerate(params['blocks']):
        h2f = {k: v.reshape(2 * v.shape[1], _HW) for k, v in h2.items()}
        out_list = _sc_block_sums(h2f, et_order,
                                  [eis[e][0] for e in et_order],
                                  [eis[e][1] for e in et_order],
                                  ones_in, dst_n, with_counts=(bi == 0))
        sums = dict(zip(et_order, out_list[:len(et_order)]))
        if bi == 0:
            cnts = dict(zip(et_order, out_list[len(et_order):]))
        last = bi == len(params['blocks']) - 1
        new = {}
        for dst, ets, bmv in (('l', ('g2l', 'l2l', 'c2l'), 512),
                              ('g', ('l2g', 'c2g'), 256),
                              ('c', ('l2c', 'g2c'), 512)):
            new[dst] = _combine(
                h2[dst], [sums[e] for e in ets], [cnts[e] for e in ets],
                [bp[e]['Wl'] for e in ets], [bp[e]['Wr'] for e in ets],
                [bp[e]['bl'] for e in ets],
                bp['n' + dst + '_g'], bp['n' + dst + '_b'],
                bm=bmv, halves_out=not last)
        h2 = new

    hg, hl, hc = h2['g'], h2['l'], h2['c']
    ctx = _sc_row_gather(hg, cause_batch.astype(jnp.int32))
    scores = _head(hc, ctx, params['head'], bm=512)
    return scores, hc, hg, hl


# speedup vs baseline: 1.0004x; 1.0000x over previous
import functools

import jax
import jax.numpy as jnp
from jax.experimental import pallas as pl
from jax.experimental.pallas import tpu as pltpu
from jax.experimental.pallas import tpu_sc as plsc

D = 512
H = 256
_HW = 128
_INTERPRET = False


def _ln(x, g, b, eps=1e-5):
    m = x.mean(-1, keepdims=True)
    v = ((x - m) ** 2).mean(-1, keepdims=True)
    return (x - m) / jnp.sqrt(v + eps) * g + b


def _gelu(x):
    return x * 0.5 * (1.0 + jax.lax.erf(x * (2.0 ** -0.5)))


def _full2d(a):
    return pl.BlockSpec(a.shape, lambda i: (0, 0))


def _fuse_proj_body(xg_ref, gt_ref, tm_ref, gate_ref, wfg_ref, wft_ref, bf_ref,
                    w1_ref, b1_ref, g1_ref, n1_ref, w2_ref, b2_ref, g2_ref,
                    n2_ref, out_ref):
    xg = xg_ref[...]
    fused = _gelu(xg @ wfg_ref[...] + gt_ref[...] @ wft_ref[...] + bf_ref[...])
    g = xg + gate_ref[0, 0] * (tm_ref[...] * fused)
    h = g @ w1_ref[...] + b1_ref[...]
    h = _gelu(_ln(h, g1_ref[...], n1_ref[...]))
    h = h @ w2_ref[...] + b2_ref[...]
    h = _ln(h, g2_ref[...], n2_ref[...])
    out_ref[...] = _aux_halves(h)


def _aux_halves(h):
    return jnp.stack([h[:, :128], h[:, 128:]])


def _fuse_proj(xg, gt, tm, gate, fuse_p, proj_p, bm):
    n = xg.shape[0]
    wfg = fuse_p['W'][:D]
    wft = fuse_p['W'][D:]
    args = (xg, gt, tm.reshape(n, 1), gate.reshape(1, 1), wfg, wft,
            fuse_p['b'].reshape(1, D),
            proj_p['W1'], proj_p['b1'].reshape(1, H),
            proj_p['g1'].reshape(1, H), proj_p['bn1'].reshape(1, H),
            proj_p['W2'], proj_p['b2'].reshape(1, H),
            proj_p['g2'].reshape(1, H), proj_p['bn2'].reshape(1, H))
    in_specs = [
        pl.BlockSpec((bm, D), lambda i: (i, 0)),
        pl.BlockSpec((bm, D), lambda i: (i, 0)),
        pl.BlockSpec((bm, 1), lambda i: (i, 0)),
        pl.BlockSpec((1, 1), lambda i: (0, 0)),
    ] + [_full2d(a) for a in args[4:]]
    return pl.pallas_call(
        _fuse_proj_body,
        grid=(n // bm,),
        in_specs=in_specs,
        out_specs=pl.BlockSpec((2, bm, _HW), lambda i: (0, i, 0)),
        out_shape=jax.ShapeDtypeStruct((2, n, _HW), jnp.float32),
        interpret=_INTERPRET,
    )(*args)


def _proj_body(x_ref, w1_ref, b1_ref, g1_ref, n1_ref, w2_ref, b2_ref, g2_ref,
               n2_ref, out_ref):
    h = x_ref[...] @ w1_ref[...] + b1_ref[...]
    h = _gelu(_ln(h, g1_ref[...], n1_ref[...]))
    h = h @ w2_ref[...] + b2_ref[...]
    h = _ln(h, g2_ref[...], n2_ref[...])
    out_ref[...] = _aux_halves(h)


def _proj(x, p, bm):
    n = x.shape[0]
    args = (x, p['W1'], p['b1'].reshape(1, H), p['g1'].reshape(1, H),
            p['bn1'].reshape(1, H), p['W2'], p['b2'].reshape(1, H),
            p['g2'].reshape(1, H), p['bn2'].reshape(1, H))
    in_specs = [pl.BlockSpec((bm, D), lambda i: (i, 0))] + \
               [_full2d(a) for a in args[1:]]
    return pl.pallas_call(
        _proj_body,
        grid=(n // bm,),
        in_specs=in_specs,
        out_specs=pl.BlockSpec((2, bm, _HW), lambda i: (0, i, 0)),
        out_shape=jax.ShapeDtypeStruct((2, n, _HW), jnp.float32),
        interpret=_INTERPRET,
    )(*args)


def _combine_body(k, halves_out, *refs):
    h_ref = refs[0]
    s_refs = refs[1:1 + k]
    c_refs = refs[1 + k:1 + 2 * k]
    wl_refs = refs[1 + 2 * k:1 + 3 * k]
    wr_refs = refs[1 + 3 * k:1 + 4 * k]
    bl_ref, g_ref, b_ref = refs[1 + 4 * k:1 + 4 * k + 3]
    out_ref = refs[-1]
    h = jnp.concatenate([h_ref[0], h_ref[1]], axis=-1)
    o = jnp.zeros_like(h)
    for s_ref, c_ref, wl_ref in zip(s_refs, c_refs, wl_refs):
        s = jnp.concatenate([s_ref[0], s_ref[1]], axis=-1)
        cnt = c_ref[0, :, 0:1] + c_ref[1, :, 0:1]
        mean = s / jnp.maximum(cnt, 1.0)
        o = o + mean @ wl_ref[...]
    wr = wr_refs[0][...]
    for r in wr_refs[1:]:
        wr = wr + r[...]
    o = o + h @ wr + bl_ref[...]
    res = _ln(_gelu(o) + h, g_ref[...], b_ref[...])
    if halves_out:
        out_ref[...] = _aux_halves(res)
    else:
        out_ref[...] = res


def _combine(h2, sums, cnts, wls, wrs, bls, g, b, bm, halves_out):
    k = len(sums)
    n = h2.shape[1]
    bl = bls[0]
    for x in bls[1:]:
        bl = bl + x
    args = ([h2] + list(sums) + list(cnts) + list(wls) + list(wrs)
            + [bl.reshape(1, H), g.reshape(1, H), b.reshape(1, H)])
    in_specs = ([pl.BlockSpec((2, bm, _HW), lambda i: (0, i, 0))]
                + [pl.BlockSpec((2, bm, _HW), lambda i: (0, i, 0))] * k
                + [pl.BlockSpec((2, bm, 8), lambda i: (0, i, 0))] * k
                + [_full2d(a) for a in args[1 + 2 * k:]])
    if halves_out:
        out_spec = pl.BlockSpec((2, bm, _HW), lambda i: (0, i, 0))
        out_shape = jax.ShapeDtypeStruct((2, n, _HW), jnp.float32)
    else:
        out_spec = pl.BlockSpec((bm, H), lambda i: (i, 0))
        out_shape = jax.ShapeDtypeStruct((n, H), jnp.float32)
    return pl.pallas_call(
        functools.partial(_combine_body, k, halves_out),
        grid=(n // bm,),
        in_specs=in_specs,
        out_specs=out_spec,
        out_shape=out_shape,
        interpret=_INTERPRET,
    )(*args)


def _head_body(hc_ref, ctx_ref, w1c_ref, w1x_ref, b1_ref, w2_ref, b2_ref,
               out_ref):
    z = _gelu(hc_ref[...] @ w1c_ref[...] + ctx_ref[...] @ w1x_ref[...]
              + b1_ref[...])
    out_ref[...] = z @ w2_ref[...] + b2_ref[0, 0]


def _head(hc, ctx, p, bm):
    n = hc.shape[0]
    w2p = jnp.pad(p['W2'], ((0, 0), (0, 127)))
    args = (hc, ctx, p['W1'][:H], p['W1'][H:], p['b1'].reshape(1, H), w2p,
            p['b2'].reshape(1, 1))
    in_specs = [pl.BlockSpec((bm, H), lambda i: (i, 0)),
                pl.BlockSpec((bm, H), lambda i: (i, 0))] + \
               [_full2d(a) for a in args[2:]]
    out = pl.pallas_call(
        _head_body,
        grid=(n // bm,),
        in_specs=in_specs,
        out_specs=pl.BlockSpec((bm, 128), lambda i: (i, 0)),
        out_shape=jax.ShapeDtypeStruct((n, 128), jnp.float32),
        interpret=_INTERPRET,
    )(*args)
    return out[:, 0]


_CHUNK = 128
_WCH = 64
_NBUF = 4


def _sc_block_sums(h2f, ets, srcs, dsts, ones_in, dst_n, with_counts):
    mesh = plsc.VectorSubcoreMesh(core_axis_name="c", subcore_axis_name="s")
    nt = len(ets)
    max_n = max(dst_n[et.split('2')[1]] for et in ets)
    out_type = [jax.ShapeDtypeStruct((2, dst_n[et.split('2')[1]], _HW),
                                     jnp.float32) for et in ets]
    if with_counts:
        out_type = out_type + [
            jax.ShapeDtypeStruct((2, dst_n[et.split('2')[1]], _HW),
                                 jnp.float32) for et in ets]
    tbls = [h2f['g'], h2f['l'], h2f['c']]
    tbl_of = {'g': 0, 'l': 1, 'c': 2}

    @functools.partial(
        pl.kernel, mesh=mesh, out_type=out_type,
        scratch_types=[
            pltpu.VMEM((_CHUNK,), jnp.int32),
            pltpu.VMEM((_CHUNK,), jnp.int32),
            pltpu.VMEM((_CHUNK,), jnp.int32),
            pltpu.VMEM((_CHUNK,), jnp.int32),
            pltpu.VMEM((1, _CHUNK), jnp.int32),
            pltpu.VMEM((1, _CHUNK), jnp.int32),
            pltpu.VMEM((_CHUNK, _HW), jnp.float32),
            pltpu.VMEM((_CHUNK, _HW), jnp.float32),
            pltpu.VMEM((_WCH, _HW), jnp.float32),
            pltpu.VMEM((_CHUNK, _HW), jnp.float32),
            pltpu.VMEM_SHARED((max_n, _HW), jnp.float32),
            pltpu.SemaphoreType.DMA,
            pltpu.SemaphoreType.DMA,
            pltpu.SemaphoreType.DMA,
            pltpu.SemaphoreType.DMA,
        ],
    )
    def k(*refs):
        tbl_refs = refs[:3]
        src_refs = refs[3:3 + nt]
        dst_refs = refs[3 + nt:3 + 2 * nt]
        ones_hbm = refs[3 + 2 * nt]
        out_refs = refs[4 + 2 * nt:4 + 3 * nt]
        no = 4 + 3 * nt + (nt if with_counts else 0)
        cnt_refs = refs[4 + 3 * nt:no]
        (srcb0, srcb1, srca0, srca1, dstb0, dstb1, rowsb0, rowsb1,
         zb, ones, acc, gsem0, gsem1, ssem0, ssem1) = refs[no:]
        c = jax.lax.axis_index("c")
        s = jax.lax.axis_index("s")
        w = c * 16 + s
        pltpu.sync_copy(ones_hbm, ones)

        def fill_z(i, _):
            for j in range(_HW // 16):
                zb[i, pl.ds(j * 16, 16)] = jnp.zeros((16,), jnp.float32)
            return 0
        jax.lax.fori_loop(0, _WCH, fill_z, 0)

        for t, et in enumerate(ets):
            skey, dkey = et.split('2')
            tref = tbl_refs[tbl_of[skey]]
            sref, dref, oref = src_refs[t], dst_refs[t], out_refs[t]
            nsrc = tref.shape[0] // 2
            e = sref.shape[0]
            n = oref.shape[1]
            rows = n // 16
            et_per_tile = e // 16
            for j in range(rows // _WCH):
                pltpu.sync_copy(zb, acc.at[pl.ds(s * rows + j * _WCH, _WCH)])
            plsc.subcore_barrier()

            base = s * et_per_tile
            nch = et_per_tile // _CHUNK

            def prep(off, sb, sa, db, rb, sem):
                pltpu.sync_copy(sref.at[pl.ds(off, _CHUNK)], sb)
                pltpu.sync_copy(dref.at[pl.ds(off, _CHUNK)], db.at[0])
                for j in range(_CHUNK // 16):
                    sa[pl.ds(j * 16, 16)] = sb[pl.ds(j * 16, 16)] + c * nsrc
                pltpu.async_copy(tref.at[sa], rb, sem)

            def fin(sa, db, rb, gsem, ssem):
                pltpu.make_async_copy(tref.at[sa], rb, gsem).wait()
                pltpu.async_copy(rb, acc.at[db.at[0]], ssem, add=True)

            def drain(db, rb, ssem):
                pltpu.make_async_copy(rb, acc.at[db.at[0]], ssem).wait()

            prep(base, srcb0, srca0, dstb0, rowsb0, gsem0)

            def body(i2, _):
                @pl.when(i2 > 0)
                def _drainb():
                    drain(dstb1, rowsb1, ssem1)
                prep(base + (2 * i2 + 1) * _CHUNK,
                     srcb1, srca1, dstb1, rowsb1, gsem1)
                fin(srca0, dstb0, rowsb0, gsem0, ssem0)

                @pl.when(i2 + 1 < nch // 2)
                def _nexta():
                    drain(dstb0, rowsb0, ssem0)
                    prep(base + (2 * i2 + 2) * _CHUNK,
                         srcb0, srca0, dstb0, rowsb0, gsem0)
                fin(srca1, dstb1, rowsb1, gsem1, ssem1)
                return 0
            jax.lax.fori_loop(0, nch // 2, body, 0)
            drain(dstb0, rowsb0, ssem0)
            drain(dstb1, rowsb1, ssem1)
            plsc.subcore_barrier()
            for j in range(rows // _WCH):
                r0 = s * rows + j * _WCH
                pltpu.sync_copy(acc.at[pl.ds(r0, _WCH)],
                                oref.at[c, pl.ds(r0, _WCH)])
            plsc.subcore_barrier()

            if not with_counts:
                continue
            cref = cnt_refs[t]
            for j in range(rows // _WCH):
                pltpu.sync_copy(zb, acc.at[pl.ds(s * rows + j * _WCH, _WCH)])
            plsc.subcore_barrier()
            ew = e // 32

            ncch = ew // _CHUNK

            def cdrain(db, ssem):
                pltpu.make_async_copy(ones, acc.at[db.at[0]], ssem).wait()

            pltpu.sync_copy(dref.at[pl.ds(w * ew, _CHUNK)], dstb0.at[0])

            def cbody(i2, _):
                @pl.when(i2 > 0)
                def _cd1():
                    cdrain(dstb1, ssem1)
                pltpu.sync_copy(
                    dref.at[pl.ds(w * ew + (2 * i2 + 1) * _CHUNK, _CHUNK)],
                    dstb1.at[0])
                pltpu.async_copy(ones, acc.at[dstb0.at[0]], ssem0, add=True)
                pltpu.async_copy(ones, acc.at[dstb1.at[0]], ssem1, add=True)

                @pl.when(i2 + 1 < ncch // 2)
                def _cnext():
                    cdrain(dstb0, ssem0)
                    pltpu.sync_copy(
                        dref.at[pl.ds(w * ew + (2 * i2 + 2) * _CHUNK,
                                      _CHUNK)], dstb0.at[0])
                return 0
            jax.lax.fori_loop(0, ncch // 2, cbody, 0)
            cdrain(dstb0, ssem0)
            cdrain(dstb1, ssem1)
            plsc.subcore_barrier()
            for j in range(rows // _WCH):
                r0 = s * rows + j * _WCH
                pltpu.sync_copy(acc.at[pl.ds(r0, _WCH)],
                                cref.at[c, pl.ds(r0, _WCH)])
            plsc.subcore_barrier()

    return k(*tbls, *srcs, *dsts, ones_in)


def _sc_row_gather(tbl, idx):
    q = idx.shape[0]
    mesh = plsc.VectorSubcoreMesh(core_axis_name="c", subcore_axis_name="s")
    qw = q // 32

    @functools.partial(
        pl.kernel, mesh=mesh,
        out_type=jax.ShapeDtypeStruct((q, 256), jnp.float32),
        scratch_types=[
            pltpu.VMEM((_CHUNK,), jnp.int32),
            pltpu.VMEM((_CHUNK, 256), jnp.float32),
            pltpu.SemaphoreType.DMA,
        ],
    )
    def k(tbl_ref, idx_ref, out_ref, idxb, rowsb, sem):
        c = jax.lax.axis_index("c")
        s = jax.lax.axis_index("s")
        w = c * 16 + s

        def body(i, _):
            off = w * qw + i * _CHUNK
            pltpu.sync_copy(idx_ref.at[pl.ds(off, _CHUNK)], idxb)
            pltpu.async_copy(tbl_ref.at[idxb], rowsb, sem).wait()
            pltpu.sync_copy(rowsb, out_ref.at[pl.ds(off, _CHUNK)])
            return 0
        jax.lax.fori_loop(0, qw // _CHUNK, body, 0)

    return k(tbl, idx)


def kernel(x_global, x_lesion, x_cause, global_txt, text_mask, ei_g2l, ei_l2g,
           ei_l2l, ei_l2c, ei_c2l, ei_g2c, ei_c2g, cause_batch, params):
    B = x_global.shape[0]
    NL = x_lesion.shape[0]
    NC = x_cause.shape[0]
    eis = {'g2l': ei_g2l, 'l2g': ei_l2g, 'l2l': ei_l2l, 'l2c': ei_l2c,
           'c2l': ei_c2l, 'g2c': ei_g2c, 'c2g': ei_c2g}
    eis = {k: v.astype(jnp.int32) for k, v in eis.items()}
    dst_n = {'g': B, 'l': NL, 'c': NC}

    et_order = list(eis.keys())

    hg2 = _fuse_proj(x_global, global_txt, text_mask, params['gate'],
                     params['fuse'], params['proj_global'], bm=256)
    hl2 = _proj(x_lesion, params['proj_lesion'], bm=512)
    hc2 = _proj(x_cause, params['proj_cause'], bm=512)

    h2 = {'g': hg2, 'l': hl2, 'c': hc2}
    ones_in = jnp.ones((_CHUNK, _HW), jnp.float32)
    for bi, bp in enumerate(params['blocks']):
        h2f = {k: v.reshape(2 * v.shape[1], _HW) for k, v in h2.items()}
        out_list = _sc_block_sums(h2f, et_order,
                                  [eis[e][0] for e in et_order],
                                  [eis[e][1] for e in et_order],
                                  ones_in, dst_n, with_counts=(bi == 0))
        sums = dict(zip(et_order, out_list[:len(et_order)]))
        if bi == 0:
            cnts = {e: a[:, :, :8]
                    for e, a in zip(et_order, out_list[len(et_order):])}
        last = bi == len(params['blocks']) - 1
        new = {}
        for dst, ets, bmv in (('l', ('g2l', 'l2l', 'c2l'), 512),
                              ('g', ('l2g', 'c2g'), 256),
                              ('c', ('l2c', 'g2c'), 512)):
            new[dst] = _combine(
                h2[dst], [sums[e] for e in ets], [cnts[e] for e in ets],
                [bp[e]['Wl'] for e in ets], [bp[e]['Wr'] for e in ets],
                [bp[e]['bl'] for e in ets],
                bp['n' + dst + '_g'], bp['n' + dst + '_b'],
                bm=bmv, halves_out=not last)
        h2 = new

    hg, hl, hc = h2['g'], h2['l'], h2['c']
    ctx = _sc_row_gather(hg, cause_batch.astype(jnp.int32))
    scores = _head(hc, ctx, params['head'], bm=512)
    return scores, hc, hg, hl


# speedup vs baseline: 1.1698x; 1.1693x over previous
import functools

import jax
import jax.numpy as jnp
from jax.experimental import pallas as pl
from jax.experimental.pallas import tpu as pltpu
from jax.experimental.pallas import tpu_sc as plsc

D = 512
H = 256
_HW = 128
_INTERPRET = False


def _ln(x, g, b, eps=1e-5):
    m = x.mean(-1, keepdims=True)
    v = ((x - m) ** 2).mean(-1, keepdims=True)
    return (x - m) / jnp.sqrt(v + eps) * g + b


def _gelu(x):
    return x * 0.5 * (1.0 + jax.lax.erf(x * (2.0 ** -0.5)))


def _full2d(a):
    return pl.BlockSpec(a.shape, lambda i: (0, 0))


def _fuse_proj_body(xg_ref, gt_ref, tm_ref, gate_ref, wfg_ref, wft_ref, bf_ref,
                    w1_ref, b1_ref, g1_ref, n1_ref, w2_ref, b2_ref, g2_ref,
                    n2_ref, out_ref):
    xg = xg_ref[...]
    fused = _gelu(xg @ wfg_ref[...] + gt_ref[...] @ wft_ref[...] + bf_ref[...])
    g = xg + gate_ref[0, 0] * (tm_ref[...] * fused)
    h = g @ w1_ref[...] + b1_ref[...]
    h = _gelu(_ln(h, g1_ref[...], n1_ref[...]))
    h = h @ w2_ref[...] + b2_ref[...]
    h = _ln(h, g2_ref[...], n2_ref[...])
    out_ref[...] = _aux_halves(h)


def _aux_halves(h):
    return jnp.stack([h[:, :128], h[:, 128:]])


def _fuse_proj(xg, gt, tm, gate, fuse_p, proj_p, bm):
    n = xg.shape[0]
    wfg = fuse_p['W'][:D]
    wft = fuse_p['W'][D:]
    args = (xg, gt, tm.reshape(n, 1), gate.reshape(1, 1), wfg, wft,
            fuse_p['b'].reshape(1, D),
            proj_p['W1'], proj_p['b1'].reshape(1, H),
            proj_p['g1'].reshape(1, H), proj_p['bn1'].reshape(1, H),
            proj_p['W2'], proj_p['b2'].reshape(1, H),
            proj_p['g2'].reshape(1, H), proj_p['bn2'].reshape(1, H))
    in_specs = [
        pl.BlockSpec((bm, D), lambda i: (i, 0)),
        pl.BlockSpec((bm, D), lambda i: (i, 0)),
        pl.BlockSpec((bm, 1), lambda i: (i, 0)),
        pl.BlockSpec((1, 1), lambda i: (0, 0)),
    ] + [_full2d(a) for a in args[4:]]
    return pl.pallas_call(
        _fuse_proj_body,
        grid=(n // bm,),
        in_specs=in_specs,
        out_specs=pl.BlockSpec((2, bm, _HW), lambda i: (0, i, 0)),
        out_shape=jax.ShapeDtypeStruct((2, n, _HW), jnp.float32),
        interpret=_INTERPRET,
    )(*args)


def _proj_body(x_ref, w1_ref, b1_ref, g1_ref, n1_ref, w2_ref, b2_ref, g2_ref,
               n2_ref, out_ref):
    h = x_ref[...] @ w1_ref[...] + b1_ref[...]
    h = _gelu(_ln(h, g1_ref[...], n1_ref[...]))
    h = h @ w2_ref[...] + b2_ref[...]
    h = _ln(h, g2_ref[...], n2_ref[...])
    out_ref[...] = _aux_halves(h)


def _proj(x, p, bm):
    n = x.shape[0]
    args = (x, p['W1'], p['b1'].reshape(1, H), p['g1'].reshape(1, H),
            p['bn1'].reshape(1, H), p['W2'], p['b2'].reshape(1, H),
            p['g2'].reshape(1, H), p['bn2'].reshape(1, H))
    in_specs = [pl.BlockSpec((bm, D), lambda i: (i, 0))] + \
               [_full2d(a) for a in args[1:]]
    return pl.pallas_call(
        _proj_body,
        grid=(n // bm,),
        in_specs=in_specs,
        out_specs=pl.BlockSpec((2, bm, _HW), lambda i: (0, i, 0)),
        out_shape=jax.ShapeDtypeStruct((2, n, _HW), jnp.float32),
        interpret=_INTERPRET,
    )(*args)


def _combine_body(k, halves_out, *refs):
    h_ref = refs[0]
    s_refs = refs[1:1 + k]
    c_refs = refs[1 + k:1 + 2 * k]
    wl_refs = refs[1 + 2 * k:1 + 3 * k]
    wr_refs = refs[1 + 3 * k:1 + 4 * k]
    bl_ref, g_ref, b_ref = refs[1 + 4 * k:1 + 4 * k + 3]
    out_ref = refs[-1]
    h = jnp.concatenate([h_ref[0], h_ref[1]], axis=-1)
    o = jnp.zeros_like(h)
    for s_ref, c_ref, wl_ref in zip(s_refs, c_refs, wl_refs):
        s = jnp.concatenate([s_ref[0], s_ref[1]], axis=-1)
        cnt = c_ref[0, :, 0:1] + c_ref[1, :, 0:1]
        mean = s / jnp.maximum(cnt, 1.0)
        o = o + mean @ wl_ref[...]
    wr = wr_refs[0][...]
    for r in wr_refs[1:]:
        wr = wr + r[...]
    o = o + h @ wr + bl_ref[...]
    res = _ln(_gelu(o) + h, g_ref[...], b_ref[...])
    if halves_out:
        out_ref[...] = _aux_halves(res)
    else:
        out_ref[...] = res


def _combine(h2, sums, cnts, wls, wrs, bls, g, b, bm, halves_out):
    k = len(sums)
    n = h2.shape[1]
    bl = bls[0]
    for x in bls[1:]:
        bl = bl + x
    args = ([h2] + list(sums) + list(cnts) + list(wls) + list(wrs)
            + [bl.reshape(1, H), g.reshape(1, H), b.reshape(1, H)])
    in_specs = ([pl.BlockSpec((2, bm, _HW), lambda i: (0, i, 0))]
                + [pl.BlockSpec((2, bm, _HW), lambda i: (0, i, 0))] * k
                + [pl.BlockSpec((2, bm, 8), lambda i: (0, i, 0))] * k
                + [_full2d(a) for a in args[1 + 2 * k:]])
    if halves_out:
        out_spec = pl.BlockSpec((2, bm, _HW), lambda i: (0, i, 0))
        out_shape = jax.ShapeDtypeStruct((2, n, _HW), jnp.float32)
    else:
        out_spec = pl.BlockSpec((bm, H), lambda i: (i, 0))
        out_shape = jax.ShapeDtypeStruct((n, H), jnp.float32)
    return pl.pallas_call(
        functools.partial(_combine_body, k, halves_out),
        grid=(n // bm,),
        in_specs=in_specs,
        out_specs=out_spec,
        out_shape=out_shape,
        interpret=_INTERPRET,
    )(*args)


def _head_body(hc_ref, ctx_ref, w1c_ref, w1x_ref, b1_ref, w2_ref, b2_ref,
               out_ref):
    z = _gelu(hc_ref[...] @ w1c_ref[...] + ctx_ref[...] @ w1x_ref[...]
              + b1_ref[...])
    out_ref[...] = z @ w2_ref[...] + b2_ref[0, 0]


def _head(hc, ctx, p, bm):
    n = hc.shape[0]
    w2p = jnp.pad(p['W2'], ((0, 0), (0, 127)))
    args = (hc, ctx, p['W1'][:H], p['W1'][H:], p['b1'].reshape(1, H), w2p,
            p['b2'].reshape(1, 1))
    in_specs = [pl.BlockSpec((bm, H), lambda i: (i, 0)),
                pl.BlockSpec((bm, H), lambda i: (i, 0))] + \
               [_full2d(a) for a in args[2:]]
    out = pl.pallas_call(
        _head_body,
        grid=(n // bm,),
        in_specs=in_specs,
        out_specs=pl.BlockSpec((bm, 128), lambda i: (i, 0)),
        out_shape=jax.ShapeDtypeStruct((n, 128), jnp.float32),
        interpret=_INTERPRET,
    )(*args)
    return out[:, 0]


_CHUNK = 128
_WCH = 64
_NBUF = 4
_CCH = 64


def _sc_block_sums(h2f, ets, srcs, dsts, ones_in, dst_n, with_counts):
    mesh = plsc.VectorSubcoreMesh(core_axis_name="c", subcore_axis_name="s")
    nt = len(ets)
    max_n = max(dst_n[et.split('2')[1]] for et in ets)
    out_type = [jax.ShapeDtypeStruct((2, dst_n[et.split('2')[1]], _HW),
                                     jnp.float32) for et in ets]
    if with_counts:
        out_type = out_type + [
            jax.ShapeDtypeStruct((2, dst_n[et.split('2')[1]], _HW),
                                 jnp.float32) for et in ets]
    tbls = [h2f['g'], h2f['l'], h2f['c']]
    tbl_of = {'g': 0, 'l': 1, 'c': 2}

    @functools.partial(
        pl.kernel, mesh=mesh, out_type=out_type,
        scratch_types=[
            pltpu.VMEM((4096,), jnp.int32),
            pltpu.VMEM((4096,), jnp.int32),
            pltpu.VMEM((_CHUNK,), jnp.int32),
            pltpu.VMEM((_CHUNK,), jnp.int32),
            pltpu.VMEM((1, _CHUNK), jnp.int32),
            pltpu.VMEM((1, _CHUNK), jnp.int32),
            pltpu.VMEM((_CHUNK, _HW), jnp.float32),
            pltpu.VMEM((_CHUNK, _HW), jnp.float32),
            pltpu.VMEM((_WCH, _HW), jnp.float32),
            pltpu.VMEM((_CCH, _HW), jnp.float32),
            pltpu.VMEM((1, _CCH), jnp.int32),
            pltpu.VMEM((1, _CCH), jnp.int32),
            pltpu.VMEM_SHARED((max_n, _HW), jnp.float32),
            pltpu.SemaphoreType.DMA,
            pltpu.SemaphoreType.DMA,
            pltpu.SemaphoreType.DMA,
            pltpu.SemaphoreType.DMA,
        ],
    )
    def k(*refs):
        tbl_refs = refs[:3]
        src_refs = refs[3:3 + nt]
        dst_refs = refs[3 + nt:3 + 2 * nt]
        ones_hbm = refs[3 + 2 * nt]
        out_refs = refs[4 + 2 * nt:4 + 3 * nt]
        no = 4 + 3 * nt + (nt if with_counts else 0)
        cnt_refs = refs[4 + 3 * nt:no]
        (fsrc, fdst, srca0, srca1, dstb0, dstb1, rowsb0, rowsb1,
         zb, ones, cdb0, cdb1, acc, gsem0, gsem1, ssem0, ssem1) = refs[no:]
        c = jax.lax.axis_index("c")
        s = jax.lax.axis_index("s")
        w = c * 16 + s
        pltpu.sync_copy(ones_hbm, ones)

        def fill_z(i, _):
            for j in range(_HW // 16):
                zb[i, pl.ds(j * 16, 16)] = jnp.zeros((16,), jnp.float32)
            return 0
        jax.lax.fori_loop(0, _WCH, fill_z, 0)

        for t, et in enumerate(ets):
            skey, dkey = et.split('2')
            tref = tbl_refs[tbl_of[skey]]
            sref, dref, oref = src_refs[t], dst_refs[t], out_refs[t]
            nsrc = tref.shape[0] // 2
            e = sref.shape[0]
            n = oref.shape[1]
            rows = n // 16
            et_per_tile = e // 16
            for j in range(rows // _WCH):
                pltpu.sync_copy(zb, acc.at[pl.ds(s * rows + j * _WCH, _WCH)])
            plsc.subcore_barrier()

            base = s * et_per_tile
            nch = et_per_tile // _CHUNK
            pltpu.sync_copy(sref.at[pl.ds(base, et_per_tile)],
                            fsrc.at[pl.ds(0, et_per_tile)])
            pltpu.sync_copy(dref.at[pl.ds(base, et_per_tile)],
                            fdst.at[pl.ds(0, et_per_tile)])

            def prep(off, sa, db, rb, sem):
                for j in range(_CHUNK // 16):
                    sa[pl.ds(j * 16, 16)] = (
                        fsrc[pl.ds(off + j * 16, 16)] + c * nsrc)
                    db[0, pl.ds(j * 16, 16)] = fdst[pl.ds(off + j * 16, 16)]
                pltpu.async_copy(tref.at[sa], rb, sem)

            def fin(sa, db, rb, gsem, ssem):
                pltpu.make_async_copy(tref.at[sa], rb, gsem).wait()
                pltpu.async_copy(rb, acc.at[db.at[0]], ssem, add=True)

            def drain(db, rb, ssem):
                pltpu.make_async_copy(rb, acc.at[db.at[0]], ssem).wait()

            prep(0, srca0, dstb0, rowsb0, gsem0)

            def body(i2, _):
                @pl.when(i2 > 0)
                def _drainb():
                    drain(dstb1, rowsb1, ssem1)
                prep((2 * i2 + 1) * _CHUNK,
                     srca1, dstb1, rowsb1, gsem1)
                fin(srca0, dstb0, rowsb0, gsem0, ssem0)

                @pl.when(i2 + 1 < nch // 2)
                def _nexta():
                    drain(dstb0, rowsb0, ssem0)
                    prep((2 * i2 + 2) * _CHUNK,
                         srca0, dstb0, rowsb0, gsem0)
                fin(srca1, dstb1, rowsb1, gsem1, ssem1)
                return 0
            jax.lax.fori_loop(0, nch // 2, body, 0)
            drain(dstb0, rowsb0, ssem0)
            drain(dstb1, rowsb1, ssem1)
            plsc.subcore_barrier()
            for j in range(rows // _WCH):
                r0 = s * rows + j * _WCH
                pltpu.sync_copy(acc.at[pl.ds(r0, _WCH)],
                                oref.at[c, pl.ds(r0, _WCH)])
            plsc.subcore_barrier()

            if not with_counts:
                continue
            cref = cnt_refs[t]
            for j in range(rows // _WCH):
                pltpu.sync_copy(zb, acc.at[pl.ds(s * rows + j * _WCH, _WCH)])
            plsc.subcore_barrier()
            ew = e // 32

            ncch = ew // _CCH
            pltpu.sync_copy(dref.at[pl.ds(w * ew, ew)],
                            fdst.at[pl.ds(0, ew)])

            def cdrain(db, ssem):
                pltpu.make_async_copy(ones, acc.at[db.at[0]], ssem).wait()

            def cprep(off, db):
                for j in range(_CCH // 16):
                    db[0, pl.ds(j * 16, 16)] = fdst[pl.ds(off + j * 16, 16)]

            cprep(0, cdb0)

            def cbody(i2, _):
                @pl.when(i2 > 0)
                def _cd1():
                    cdrain(cdb1, ssem1)
                cprep((2 * i2 + 1) * _CCH, cdb1)
                pltpu.async_copy(ones, acc.at[cdb0.at[0]], ssem0, add=True)
                pltpu.async_copy(ones, acc.at[cdb1.at[0]], ssem1, add=True)

                @pl.when(i2 + 1 < ncch // 2)
                def _cnext():
                    cdrain(cdb0, ssem0)
                    cprep((2 * i2 + 2) * _CCH, cdb0)
                return 0
            jax.lax.fori_loop(0, ncch // 2, cbody, 0)
            cdrain(cdb0, ssem0)
            cdrain(cdb1, ssem1)
            plsc.subcore_barrier()
            for j in range(rows // _WCH):
                r0 = s * rows + j * _WCH
                pltpu.sync_copy(acc.at[pl.ds(r0, _WCH)],
                                cref.at[c, pl.ds(r0, _WCH)])
            plsc.subcore_barrier()

    return k(*tbls, *srcs, *dsts, ones_in)


def _sc_row_gather(tbl, idx):
    q = idx.shape[0]
    mesh = plsc.VectorSubcoreMesh(core_axis_name="c", subcore_axis_name="s")
    qw = q // 32

    @functools.partial(
        pl.kernel, mesh=mesh,
        out_type=jax.ShapeDtypeStruct((q, 256), jnp.float32),
        scratch_types=[
            pltpu.VMEM((_CHUNK,), jnp.int32),
            pltpu.VMEM((_CHUNK, 256), jnp.float32),
            pltpu.SemaphoreType.DMA,
        ],
    )
    def k(tbl_ref, idx_ref, out_ref, idxb, rowsb, sem):
        c = jax.lax.axis_index("c")
        s = jax.lax.axis_index("s")
        w = c * 16 + s

        def body(i, _):
            off = w * qw + i * _CHUNK
            pltpu.sync_copy(idx_ref.at[pl.ds(off, _CHUNK)], idxb)
            pltpu.async_copy(tbl_ref.at[idxb], rowsb, sem).wait()
            pltpu.sync_copy(rowsb, out_ref.at[pl.ds(off, _CHUNK)])
            return 0
        jax.lax.fori_loop(0, qw // _CHUNK, body, 0)

    return k(tbl, idx)


def kernel(x_global, x_lesion, x_cause, global_txt, text_mask, ei_g2l, ei_l2g,
           ei_l2l, ei_l2c, ei_c2l, ei_g2c, ei_c2g, cause_batch, params):
    B = x_global.shape[0]
    NL = x_lesion.shape[0]
    NC = x_cause.shape[0]
    eis = {'g2l': ei_g2l, 'l2g': ei_l2g, 'l2l': ei_l2l, 'l2c': ei_l2c,
           'c2l': ei_c2l, 'g2c': ei_g2c, 'c2g': ei_c2g}
    eis = {k: v.astype(jnp.int32) for k, v in eis.items()}
    dst_n = {'g': B, 'l': NL, 'c': NC}

    et_order = list(eis.keys())

    hg2 = _fuse_proj(x_global, global_txt, text_mask, params['gate'],
                     params['fuse'], params['proj_global'], bm=256)
    hl2 = _proj(x_lesion, params['proj_lesion'], bm=512)
    hc2 = _proj(x_cause, params['proj_cause'], bm=512)

    h2 = {'g': hg2, 'l': hl2, 'c': hc2}
    ones_in = jnp.ones((_CCH, _HW), jnp.float32)
    for bi, bp in enumerate(params['blocks']):
        h2f = {k: v.reshape(2 * v.shape[1], _HW) for k, v in h2.items()}
        out_list = _sc_block_sums(h2f, et_order,
                                  [eis[e][0] for e in et_order],
                                  [eis[e][1] for e in et_order],
                                  ones_in, dst_n, with_counts=(bi == 0))
        sums = dict(zip(et_order, out_list[:len(et_order)]))
        if bi == 0:
            cnts = {e: a[:, :, :8]
                    for e, a in zip(et_order, out_list[len(et_order):])}
        last = bi == len(params['blocks']) - 1
        new = {}
        for dst, ets, bmv in (('l', ('g2l', 'l2l', 'c2l'), 512),
                              ('g', ('l2g', 'c2g'), 256),
                              ('c', ('l2c', 'g2c'), 512)):
            new[dst] = _combine(
                h2[dst], [sums[e] for e in ets], [cnts[e] for e in ets],
                [bp[e]['Wl'] for e in ets], [bp[e]['Wr'] for e in ets],
                [bp[e]['bl'] for e in ets],
                bp['n' + dst + '_g'], bp['n' + dst + '_b'],
                bm=bmv, halves_out=not last)
        h2 = new

    hg, hl, hc = h2['g'], h2['l'], h2['c']
    ctx = _sc_row_gather(hg, cause_batch.astype(jnp.int32))
    scores = _head(hc, ctx, params['head'], bm=512)
    return scores, hc, hg, hl


# speedup vs baseline: 1.1890x; 1.0165x over previous
import functools

import jax
import jax.numpy as jnp
from jax.experimental import pallas as pl
from jax.experimental.pallas import tpu as pltpu
from jax.experimental.pallas import tpu_sc as plsc

D = 512
H = 256
_HW = 128
_INTERPRET = False


def _ln(x, g, b, eps=1e-5):
    m = x.mean(-1, keepdims=True)
    v = ((x - m) ** 2).mean(-1, keepdims=True)
    return (x - m) / jnp.sqrt(v + eps) * g + b


def _gelu(x):
    return x * 0.5 * (1.0 + jax.lax.erf(x * (2.0 ** -0.5)))


def _full2d(a):
    return pl.BlockSpec(a.shape, lambda i: (0, 0))


def _fuse_proj_body(xg_ref, gt_ref, tm_ref, gate_ref, wfg_ref, wft_ref, bf_ref,
                    w1_ref, b1_ref, g1_ref, n1_ref, w2_ref, b2_ref, g2_ref,
                    n2_ref, out_ref):
    xg = xg_ref[...]
    fused = _gelu(xg @ wfg_ref[...] + gt_ref[...] @ wft_ref[...] + bf_ref[...])
    g = xg + gate_ref[0, 0] * (tm_ref[...] * fused)
    h = g @ w1_ref[...] + b1_ref[...]
    h = _gelu(_ln(h, g1_ref[...], n1_ref[...]))
    h = h @ w2_ref[...] + b2_ref[...]
    h = _ln(h, g2_ref[...], n2_ref[...])
    out_ref[...] = _aux_halves(h)


def _aux_halves(h):
    return jnp.stack([h[:, :128], h[:, 128:]])


def _fuse_proj(xg, gt, tm, gate, fuse_p, proj_p, bm):
    n = xg.shape[0]
    wfg = fuse_p['W'][:D]
    wft = fuse_p['W'][D:]
    args = (xg, gt, tm.reshape(n, 1), gate.reshape(1, 1), wfg, wft,
            fuse_p['b'].reshape(1, D),
            proj_p['W1'], proj_p['b1'].reshape(1, H),
            proj_p['g1'].reshape(1, H), proj_p['bn1'].reshape(1, H),
            proj_p['W2'], proj_p['b2'].reshape(1, H),
            proj_p['g2'].reshape(1, H), proj_p['bn2'].reshape(1, H))
    in_specs = [
        pl.BlockSpec((bm, D), lambda i: (i, 0)),
        pl.BlockSpec((bm, D), lambda i: (i, 0)),
        pl.BlockSpec((bm, 1), lambda i: (i, 0)),
        pl.BlockSpec((1, 1), lambda i: (0, 0)),
    ] + [_full2d(a) for a in args[4:]]
    return pl.pallas_call(
        _fuse_proj_body,
        grid=(n // bm,),
        in_specs=in_specs,
        out_specs=pl.BlockSpec((2, bm, _HW), lambda i: (0, i, 0)),
        out_shape=jax.ShapeDtypeStruct((2, n, _HW), jnp.float32),
        interpret=_INTERPRET,
    )(*args)


def _proj_body(x_ref, w1_ref, b1_ref, g1_ref, n1_ref, w2_ref, b2_ref, g2_ref,
               n2_ref, out_ref):
    h = x_ref[...] @ w1_ref[...] + b1_ref[...]
    h = _gelu(_ln(h, g1_ref[...], n1_ref[...]))
    h = h @ w2_ref[...] + b2_ref[...]
    h = _ln(h, g2_ref[...], n2_ref[...])
    out_ref[...] = _aux_halves(h)


def _proj(x, p, bm):
    n = x.shape[0]
    args = (x, p['W1'], p['b1'].reshape(1, H), p['g1'].reshape(1, H),
            p['bn1'].reshape(1, H), p['W2'], p['b2'].reshape(1, H),
            p['g2'].reshape(1, H), p['bn2'].reshape(1, H))
    in_specs = [pl.BlockSpec((bm, D), lambda i: (i, 0))] + \
               [_full2d(a) for a in args[1:]]
    return pl.pallas_call(
        _proj_body,
        grid=(n // bm,),
        in_specs=in_specs,
        out_specs=pl.BlockSpec((2, bm, _HW), lambda i: (0, i, 0)),
        out_shape=jax.ShapeDtypeStruct((2, n, _HW), jnp.float32),
        interpret=_INTERPRET,
    )(*args)


def _combine_body(k, halves_out, *refs):
    h_ref = refs[0]
    s_refs = refs[1:1 + k]
    c_refs = refs[1 + k:1 + 2 * k]
    wl_refs = refs[1 + 2 * k:1 + 3 * k]
    wr_refs = refs[1 + 3 * k:1 + 4 * k]
    bl_ref, g_ref, b_ref = refs[1 + 4 * k:1 + 4 * k + 3]
    out_ref = refs[-1]
    h = jnp.concatenate([h_ref[0], h_ref[1]], axis=-1)
    o = jnp.zeros_like(h)
    for s_ref, c_ref, wl_ref in zip(s_refs, c_refs, wl_refs):
        s = jnp.concatenate([s_ref[0], s_ref[1]], axis=-1)
        cnt = c_ref[0, :, 0:1] + c_ref[1, :, 0:1]
        mean = s / jnp.maximum(cnt, 1.0)
        o = o + mean @ wl_ref[...]
    wr = wr_refs[0][...]
    for r in wr_refs[1:]:
        wr = wr + r[...]
    o = o + h @ wr + bl_ref[...]
    res = _ln(_gelu(o) + h, g_ref[...], b_ref[...])
    if halves_out:
        out_ref[...] = _aux_halves(res)
    else:
        out_ref[...] = res


def _combine(h2, sums, cnts, wls, wrs, bls, g, b, bm, halves_out):
    k = len(sums)
    n = h2.shape[1]
    bl = bls[0]
    for x in bls[1:]:
        bl = bl + x
    args = ([h2] + list(sums) + list(cnts) + list(wls) + list(wrs)
            + [bl.reshape(1, H), g.reshape(1, H), b.reshape(1, H)])
    in_specs = ([pl.BlockSpec((2, bm, _HW), lambda i: (0, i, 0))]
                + [pl.BlockSpec((2, bm, _HW), lambda i: (0, i, 0))] * k
                + [pl.BlockSpec((2, bm, 8), lambda i: (0, i, 0))] * k
                + [_full2d(a) for a in args[1 + 2 * k:]])
    if halves_out:
        out_spec = pl.BlockSpec((2, bm, _HW), lambda i: (0, i, 0))
        out_shape = jax.ShapeDtypeStruct((2, n, _HW), jnp.float32)
    else:
        out_spec = pl.BlockSpec((bm, H), lambda i: (i, 0))
        out_shape = jax.ShapeDtypeStruct((n, H), jnp.float32)
    return pl.pallas_call(
        functools.partial(_combine_body, k, halves_out),
        grid=(n // bm,),
        in_specs=in_specs,
        out_specs=out_spec,
        out_shape=out_shape,
        interpret=_INTERPRET,
    )(*args)


def _head_body(hc_ref, ctx_ref, w1c_ref, w1x_ref, b1_ref, w2_ref, b2_ref,
               out_ref):
    z = _gelu(hc_ref[...] @ w1c_ref[...] + ctx_ref[...] @ w1x_ref[...]
              + b1_ref[...])
    out_ref[...] = z @ w2_ref[...] + b2_ref[0, 0]


def _head(hc, ctx, p, bm):
    n = hc.shape[0]
    w2p = jnp.pad(p['W2'], ((0, 0), (0, 127)))
    args = (hc, ctx, p['W1'][:H], p['W1'][H:], p['b1'].reshape(1, H), w2p,
            p['b2'].reshape(1, 1))
    in_specs = [pl.BlockSpec((bm, H), lambda i: (i, 0)),
                pl.BlockSpec((bm, H), lambda i: (i, 0))] + \
               [_full2d(a) for a in args[2:]]
    out = pl.pallas_call(
        _head_body,
        grid=(n // bm,),
        in_specs=in_specs,
        out_specs=pl.BlockSpec((bm, 128), lambda i: (i, 0)),
        out_shape=jax.ShapeDtypeStruct((n, 128), jnp.float32),
        interpret=_INTERPRET,
    )(*args)
    return out[:, 0]


_CHUNK = 128
_WCH = 64
_NBUF = 4
_CCH = 64


def _sc_block_sums(h2f, ets, srcs, dsts, ones_in, dst_n, with_counts):
    mesh = plsc.VectorSubcoreMesh(core_axis_name="c", subcore_axis_name="s")
    nt = len(ets)
    max_n = max(dst_n[et.split('2')[1]] for et in ets)
    out_type = [jax.ShapeDtypeStruct((2, dst_n[et.split('2')[1]], _HW),
                                     jnp.float32) for et in ets]
    if with_counts:
        out_type = out_type + [
            jax.ShapeDtypeStruct((2, dst_n[et.split('2')[1]], _HW),
                                 jnp.float32) for et in ets]
    tbls = [h2f['g'], h2f['l'], h2f['c']]
    tbl_of = {'g': 0, 'l': 1, 'c': 2}

    @functools.partial(
        pl.kernel, mesh=mesh, out_type=out_type,
        scratch_types=[
            pltpu.VMEM((4096,), jnp.int32),
            pltpu.VMEM((4096,), jnp.int32),
            pltpu.VMEM((_CHUNK,), jnp.int32),
            pltpu.VMEM((_CHUNK,), jnp.int32),
            pltpu.VMEM((1, _CHUNK), jnp.int32),
            pltpu.VMEM((1, _CHUNK), jnp.int32),
            pltpu.VMEM((_CHUNK, _HW), jnp.float32),
            pltpu.VMEM((_CHUNK, _HW), jnp.float32),
            pltpu.VMEM((_WCH, _HW), jnp.float32),
            pltpu.VMEM((_CCH, _HW), jnp.float32),
            pltpu.VMEM((1, _CCH), jnp.int32),
            pltpu.VMEM((1, _CCH), jnp.int32),
            pltpu.VMEM_SHARED((max_n, _HW), jnp.float32),
            pltpu.SemaphoreType.DMA,
            pltpu.SemaphoreType.DMA,
            pltpu.SemaphoreType.DMA,
            pltpu.SemaphoreType.DMA,
        ],
    )
    def k(*refs):
        tbl_refs = refs[:3]
        src_refs = refs[3:3 + nt]
        dst_refs = refs[3 + nt:3 + 2 * nt]
        ones_hbm = refs[3 + 2 * nt]
        out_refs = refs[4 + 2 * nt:4 + 3 * nt]
        no = 4 + 3 * nt + (nt if with_counts else 0)
        cnt_refs = refs[4 + 3 * nt:no]
        (fsrc, fdst, srca0, srca1, dstb0, dstb1, rowsb0, rowsb1,
         zb, ones, cdb0, cdb1, acc, gsem0, gsem1, ssem0, ssem1) = refs[no:]
        c = jax.lax.axis_index("c")
        s = jax.lax.axis_index("s")
        w = c * 16 + s
        pltpu.sync_copy(ones_hbm, ones)

        def fill_z(i, _):
            for j in range(_HW // 16):
                zb[i, pl.ds(j * 16, 16)] = jnp.zeros((16,), jnp.float32)
            return 0
        jax.lax.fori_loop(0, _WCH, fill_z, 0)

        for t, et in enumerate(ets):
            skey, dkey = et.split('2')
            tref = tbl_refs[tbl_of[skey]]
            sref, dref, oref = src_refs[t], dst_refs[t], out_refs[t]
            nsrc = tref.shape[0] // 2
            e = sref.shape[0]
            n = oref.shape[1]
            rows = n // 16
            et_per_tile = e // 16
            zhs = [pltpu.async_copy(
                zb, acc.at[pl.ds(s * rows + j * _WCH, _WCH)], gsem0)
                for j in range(rows // _WCH)]
            for h in zhs:
                h.wait()
            plsc.subcore_barrier()

            base = s * et_per_tile
            nch = et_per_tile // _CHUNK
            pltpu.sync_copy(sref.at[pl.ds(base, et_per_tile)],
                            fsrc.at[pl.ds(0, et_per_tile)])
            pltpu.sync_copy(dref.at[pl.ds(base, et_per_tile)],
                            fdst.at[pl.ds(0, et_per_tile)])

            def prep(off, sa, db, rb, sem):
                for j in range(_CHUNK // 16):
                    sa[pl.ds(j * 16, 16)] = (
                        fsrc[pl.ds(off + j * 16, 16)] + c * nsrc)
                    db[0, pl.ds(j * 16, 16)] = fdst[pl.ds(off + j * 16, 16)]
                pltpu.async_copy(tref.at[sa], rb, sem)

            def fin(sa, db, rb, gsem, ssem):
                pltpu.make_async_copy(tref.at[sa], rb, gsem).wait()
                pltpu.async_copy(rb, acc.at[db.at[0]], ssem, add=True)

            def drain(db, rb, ssem):
                pltpu.make_async_copy(rb, acc.at[db.at[0]], ssem).wait()

            prep(0, srca0, dstb0, rowsb0, gsem0)

            def body(i2, _):
                @pl.when(i2 > 0)
                def _drainb():
                    drain(dstb1, rowsb1, ssem1)
                prep((2 * i2 + 1) * _CHUNK,
                     srca1, dstb1, rowsb1, gsem1)
                fin(srca0, dstb0, rowsb0, gsem0, ssem0)

                @pl.when(i2 + 1 < nch // 2)
                def _nexta():
                    drain(dstb0, rowsb0, ssem0)
                    prep((2 * i2 + 2) * _CHUNK,
                         srca0, dstb0, rowsb0, gsem0)
                fin(srca1, dstb1, rowsb1, gsem1, ssem1)
                return 0
            jax.lax.fori_loop(0, nch // 2, body, 0)
            drain(dstb0, rowsb0, ssem0)
            drain(dstb1, rowsb1, ssem1)
            plsc.subcore_barrier()
            whs = [pltpu.async_copy(
                acc.at[pl.ds(s * rows + j * _WCH, _WCH)],
                oref.at[c, pl.ds(s * rows + j * _WCH, _WCH)], gsem0)
                for j in range(rows // _WCH)]
            for h in whs:
                h.wait()
            plsc.subcore_barrier()

            if not with_counts:
                continue
            cref = cnt_refs[t]
            czhs = [pltpu.async_copy(
                zb, acc.at[pl.ds(s * rows + j * _WCH, _WCH)], gsem0)
                for j in range(rows // _WCH)]
            for h in czhs:
                h.wait()
            plsc.subcore_barrier()
            ew = e // 32

            ncch = ew // _CCH
            pltpu.sync_copy(dref.at[pl.ds(w * ew, ew)],
                            fdst.at[pl.ds(0, ew)])

            def cdrain(db, ssem):
                pltpu.make_async_copy(ones, acc.at[db.at[0]], ssem).wait()

            def cprep(off, db):
                for j in range(_CCH // 16):
                    db[0, pl.ds(j * 16, 16)] = fdst[pl.ds(off + j * 16, 16)]

            cprep(0, cdb0)

            def cbody(i2, _):
                @pl.when(i2 > 0)
                def _cd1():
                    cdrain(cdb1, ssem1)
                cprep((2 * i2 + 1) * _CCH, cdb1)
                pltpu.async_copy(ones, acc.at[cdb0.at[0]], ssem0, add=True)
                pltpu.async_copy(ones, acc.at[cdb1.at[0]], ssem1, add=True)

                @pl.when(i2 + 1 < ncch // 2)
                def _cnext():
                    cdrain(cdb0, ssem0)
                    cprep((2 * i2 + 2) * _CCH, cdb0)
                return 0
            jax.lax.fori_loop(0, ncch // 2, cbody, 0)
            cdrain(cdb0, ssem0)
            cdrain(cdb1, ssem1)
            plsc.subcore_barrier()
            cwhs = [pltpu.async_copy(
                acc.at[pl.ds(s * rows + j * _WCH, _WCH)],
                cref.at[c, pl.ds(s * rows + j * _WCH, _WCH)], gsem0)
                for j in range(rows // _WCH)]
            for h in cwhs:
                h.wait()
            plsc.subcore_barrier()

    return k(*tbls, *srcs, *dsts, ones_in)


def _sc_row_gather(tbl, idx):
    q = idx.shape[0]
    mesh = plsc.VectorSubcoreMesh(core_axis_name="c", subcore_axis_name="s")
    qw = q // 32

    @functools.partial(
        pl.kernel, mesh=mesh,
        out_type=jax.ShapeDtypeStruct((q, 256), jnp.float32),
        scratch_types=[
            pltpu.VMEM((_CHUNK,), jnp.int32),
            pltpu.VMEM((_CHUNK, 256), jnp.float32),
            pltpu.SemaphoreType.DMA,
        ],
    )
    def k(tbl_ref, idx_ref, out_ref, idxb, rowsb, sem):
        c = jax.lax.axis_index("c")
        s = jax.lax.axis_index("s")
        w = c * 16 + s

        def body(i, _):
            off = w * qw + i * _CHUNK
            pltpu.sync_copy(idx_ref.at[pl.ds(off, _CHUNK)], idxb)
            pltpu.async_copy(tbl_ref.at[idxb], rowsb, sem).wait()
            pltpu.sync_copy(rowsb, out_ref.at[pl.ds(off, _CHUNK)])
            return 0
        jax.lax.fori_loop(0, qw // _CHUNK, body, 0)

    return k(tbl, idx)


def kernel(x_global, x_lesion, x_cause, global_txt, text_mask, ei_g2l, ei_l2g,
           ei_l2l, ei_l2c, ei_c2l, ei_g2c, ei_c2g, cause_batch, params):
    B = x_global.shape[0]
    NL = x_lesion.shape[0]
    NC = x_cause.shape[0]
    eis = {'g2l': ei_g2l, 'l2g': ei_l2g, 'l2l': ei_l2l, 'l2c': ei_l2c,
           'c2l': ei_c2l, 'g2c': ei_g2c, 'c2g': ei_c2g}
    eis = {k: v.astype(jnp.int32) for k, v in eis.items()}
    dst_n = {'g': B, 'l': NL, 'c': NC}

    et_order = list(eis.keys())

    hg2 = _fuse_proj(x_global, global_txt, text_mask, params['gate'],
                     params['fuse'], params['proj_global'], bm=256)
    hl2 = _proj(x_lesion, params['proj_lesion'], bm=512)
    hc2 = _proj(x_cause, params['proj_cause'], bm=512)

    h2 = {'g': hg2, 'l': hl2, 'c': hc2}
    ones_in = jnp.ones((_CCH, _HW), jnp.float32)
    for bi, bp in enumerate(params['blocks']):
        h2f = {k: v.reshape(2 * v.shape[1], _HW) for k, v in h2.items()}
        out_list = _sc_block_sums(h2f, et_order,
                                  [eis[e][0] for e in et_order],
                                  [eis[e][1] for e in et_order],
                                  ones_in, dst_n, with_counts=(bi == 0))
        sums = dict(zip(et_order, out_list[:len(et_order)]))
        if bi == 0:
            cnts = {e: a[:, :, :8]
                    for e, a in zip(et_order, out_list[len(et_order):])}
        last = bi == len(params['blocks']) - 1
        new = {}
        for dst, ets, bmv in (('l', ('g2l', 'l2l', 'c2l'), 512),
                              ('g', ('l2g', 'c2g'), 256),
                              ('c', ('l2c', 'g2c'), 512)):
            new[dst] = _combine(
                h2[dst], [sums[e] for e in ets], [cnts[e] for e in ets],
                [bp[e]['Wl'] for e in ets], [bp[e]['Wr'] for e in ets],
                [bp[e]['bl'] for e in ets],
                bp['n' + dst + '_g'], bp['n' + dst + '_b'],
                bm=bmv, halves_out=not last)
        h2 = new

    hg, hl, hc = h2['g'], h2['l'], h2['c']
    ctx = _sc_row_gather(hg, cause_batch.astype(jnp.int32))
    scores = _head(hc, ctx, params['head'], bm=512)
    return scores, hc, hg, hl


# speedup vs baseline: 1.1896x; 1.0005x over previous
import functools

import jax
import jax.numpy as jnp
from jax.experimental import pallas as pl
from jax.experimental.pallas import tpu as pltpu
from jax.experimental.pallas import tpu_sc as plsc

D = 512
H = 256
_HW = 128


def _ln(x, g, b, eps=1e-5):
    m = x.mean(-1, keepdims=True)
    v = ((x - m) ** 2).mean(-1, keepdims=True)
    return (x - m) / jnp.sqrt(v + eps) * g + b


def _gelu(x):
    return x * 0.5 * (1.0 + jax.lax.erf(x * (2.0 ** -0.5)))


def _full2d(a):
    return pl.BlockSpec(a.shape, lambda i: (0, 0))


def _fuse_proj_body(xg_ref, gt_ref, tm_ref, gate_ref, wfg_ref, wft_ref, bf_ref,
                    w1_ref, b1_ref, g1_ref, n1_ref, w2_ref, b2_ref, g2_ref,
                    n2_ref, out_ref):
    xg = xg_ref[...]
    fused = _gelu(xg @ wfg_ref[...] + gt_ref[...] @ wft_ref[...] + bf_ref[...])
    g = xg + gate_ref[0, 0] * (tm_ref[...] * fused)
    h = g @ w1_ref[...] + b1_ref[...]
    h = _gelu(_ln(h, g1_ref[...], n1_ref[...]))
    h = h @ w2_ref[...] + b2_ref[...]
    h = _ln(h, g2_ref[...], n2_ref[...])
    out_ref[...] = _aux_halves(h)


def _aux_halves(h):
    return jnp.stack([h[:, :128], h[:, 128:]])


def _fuse_proj(xg, gt, tm, gate, fuse_p, proj_p, bm):
    n = xg.shape[0]
    wfg = fuse_p['W'][:D]
    wft = fuse_p['W'][D:]
    args = (xg, gt, tm.reshape(n, 1), gate.reshape(1, 1), wfg, wft,
            fuse_p['b'].reshape(1, D),
            proj_p['W1'], proj_p['b1'].reshape(1, H),
            proj_p['g1'].reshape(1, H), proj_p['bn1'].reshape(1, H),
            proj_p['W2'], proj_p['b2'].reshape(1, H),
            proj_p['g2'].reshape(1, H), proj_p['bn2'].reshape(1, H))
    in_specs = [
        pl.BlockSpec((bm, D), lambda i: (i, 0)),
        pl.BlockSpec((bm, D), lambda i: (i, 0)),
        pl.BlockSpec((bm, 1), lambda i: (i, 0)),
        pl.BlockSpec((1, 1), lambda i: (0, 0)),
    ] + [_full2d(a) for a in args[4:]]
    return pl.pallas_call(
        _fuse_proj_body,
        grid=(n // bm,),
        in_specs=in_specs,
        out_specs=pl.BlockSpec((2, bm, _HW), lambda i: (0, i, 0)),
        out_shape=jax.ShapeDtypeStruct((2, n, _HW), jnp.float32),
    )(*args)


def _proj_body(x_ref, w1_ref, b1_ref, g1_ref, n1_ref, w2_ref, b2_ref, g2_ref,
               n2_ref, out_ref):
    h = x_ref[...] @ w1_ref[...] + b1_ref[...]
    h = _gelu(_ln(h, g1_ref[...], n1_ref[...]))
    h = h @ w2_ref[...] + b2_ref[...]
    h = _ln(h, g2_ref[...], n2_ref[...])
    out_ref[...] = _aux_halves(h)


def _proj(x, p, bm):
    n = x.shape[0]
    args = (x, p['W1'], p['b1'].reshape(1, H), p['g1'].reshape(1, H),
            p['bn1'].reshape(1, H), p['W2'], p['b2'].reshape(1, H),
            p['g2'].reshape(1, H), p['bn2'].reshape(1, H))
    in_specs = [pl.BlockSpec((bm, D), lambda i: (i, 0))] + \
               [_full2d(a) for a in args[1:]]
    return pl.pallas_call(
        _proj_body,
        grid=(n // bm,),
        in_specs=in_specs,
        out_specs=pl.BlockSpec((2, bm, _HW), lambda i: (0, i, 0)),
        out_shape=jax.ShapeDtypeStruct((2, n, _HW), jnp.float32),
    )(*args)


def _combine_body(k, halves_out, *refs):
    h_ref = refs[0]
    s_refs = refs[1:1 + k]
    c_refs = refs[1 + k:1 + 2 * k]
    wl_refs = refs[1 + 2 * k:1 + 3 * k]
    wr_refs = refs[1 + 3 * k:1 + 4 * k]
    bl_ref, g_ref, b_ref = refs[1 + 4 * k:1 + 4 * k + 3]
    out_ref = refs[-1]
    h = jnp.concatenate([h_ref[0], h_ref[1]], axis=-1)
    o = jnp.zeros_like(h)
    for s_ref, c_ref, wl_ref in zip(s_refs, c_refs, wl_refs):
        s = jnp.concatenate([s_ref[0], s_ref[1]], axis=-1)
        cnt = c_ref[0, :, 0:1] + c_ref[1, :, 0:1]
        mean = s / jnp.maximum(cnt, 1.0)
        o = o + mean @ wl_ref[...]
    wr = wr_refs[0][...]
    for r in wr_refs[1:]:
        wr = wr + r[...]
    o = o + h @ wr + bl_ref[...]
    res = _ln(_gelu(o) + h, g_ref[...], b_ref[...])
    if halves_out:
        out_ref[...] = _aux_halves(res)
    else:
        out_ref[...] = res


def _combine(h2, sums, cnts, wls, wrs, bls, g, b, bm, halves_out):
    k = len(sums)
    n = h2.shape[1]
    bl = bls[0]
    for x in bls[1:]:
        bl = bl + x
    args = ([h2] + list(sums) + list(cnts) + list(wls) + list(wrs)
            + [bl.reshape(1, H), g.reshape(1, H), b.reshape(1, H)])
    in_specs = ([pl.BlockSpec((2, bm, _HW), lambda i: (0, i, 0))]
                + [pl.BlockSpec((2, bm, _HW), lambda i: (0, i, 0))] * k
                + [pl.BlockSpec((2, bm, 8), lambda i: (0, i, 0))] * k
                + [_full2d(a) for a in args[1 + 2 * k:]])
    if halves_out:
        out_spec = pl.BlockSpec((2, bm, _HW), lambda i: (0, i, 0))
        out_shape = jax.ShapeDtypeStruct((2, n, _HW), jnp.float32)
    else:
        out_spec = pl.BlockSpec((bm, H), lambda i: (i, 0))
        out_shape = jax.ShapeDtypeStruct((n, H), jnp.float32)
    return pl.pallas_call(
        functools.partial(_combine_body, k, halves_out),
        grid=(n // bm,),
        in_specs=in_specs,
        out_specs=out_spec,
        out_shape=out_shape,
    )(*args)


def _head_body(hc_ref, ctx_ref, w1c_ref, w1x_ref, b1_ref, w2_ref, b2_ref,
               out_ref):
    z = _gelu(hc_ref[...] @ w1c_ref[...] + ctx_ref[...] @ w1x_ref[...]
              + b1_ref[...])
    out_ref[...] = z @ w2_ref[...] + b2_ref[0, 0]


def _head(hc, ctx, p, bm):
    n = hc.shape[0]
    w2p = jnp.pad(p['W2'], ((0, 0), (0, 127)))
    args = (hc, ctx, p['W1'][:H], p['W1'][H:], p['b1'].reshape(1, H), w2p,
            p['b2'].reshape(1, 1))
    in_specs = [pl.BlockSpec((bm, H), lambda i: (i, 0)),
                pl.BlockSpec((bm, H), lambda i: (i, 0))] + \
               [_full2d(a) for a in args[2:]]
    out = pl.pallas_call(
        _head_body,
        grid=(n // bm,),
        in_specs=in_specs,
        out_specs=pl.BlockSpec((bm, 128), lambda i: (i, 0)),
        out_shape=jax.ShapeDtypeStruct((n, 128), jnp.float32),
    )(*args)
    return out[:, 0]


_CHUNK = 128
_WCH = 64
_CCH = 64


def _sc_block_sums(h2f, ets, srcs, dsts, ones_in, dst_n, with_counts):
    mesh = plsc.VectorSubcoreMesh(core_axis_name="c", subcore_axis_name="s")
    nt = len(ets)
    max_n = max(dst_n[et.split('2')[1]] for et in ets)
    out_type = [jax.ShapeDtypeStruct((2, dst_n[et.split('2')[1]], _HW),
                                     jnp.float32) for et in ets]
    if with_counts:
        out_type = out_type + [
            jax.ShapeDtypeStruct((2, dst_n[et.split('2')[1]], _HW),
                                 jnp.float32) for et in ets]
    tbls = [h2f['g'], h2f['l'], h2f['c']]
    tbl_of = {'g': 0, 'l': 1, 'c': 2}

    @functools.partial(
        pl.kernel, mesh=mesh, out_type=out_type,
        scratch_types=[
            pltpu.VMEM((4096,), jnp.int32),
            pltpu.VMEM((4096,), jnp.int32),
            pltpu.VMEM((_CHUNK,), jnp.int32),
            pltpu.VMEM((_CHUNK,), jnp.int32),
            pltpu.VMEM((1, _CHUNK), jnp.int32),
            pltpu.VMEM((1, _CHUNK), jnp.int32),
            pltpu.VMEM((_CHUNK, _HW), jnp.float32),
            pltpu.VMEM((_CHUNK, _HW), jnp.float32),
            pltpu.VMEM((_WCH, _HW), jnp.float32),
            pltpu.VMEM((_CCH, _HW), jnp.float32),
            pltpu.VMEM((1, _CCH), jnp.int32),
            pltpu.VMEM((1, _CCH), jnp.int32),
            pltpu.VMEM_SHARED((max_n, _HW), jnp.float32),
            pltpu.SemaphoreType.DMA,
            pltpu.SemaphoreType.DMA,
            pltpu.SemaphoreType.DMA,
            pltpu.SemaphoreType.DMA,
        ],
    )
    def k(*refs):
        tbl_refs = refs[:3]
        src_refs = refs[3:3 + nt]
        dst_refs = refs[3 + nt:3 + 2 * nt]
        ones_hbm = refs[3 + 2 * nt]
        out_refs = refs[4 + 2 * nt:4 + 3 * nt]
        no = 4 + 3 * nt + (nt if with_counts else 0)
        cnt_refs = refs[4 + 3 * nt:no]
        (fsrc, fdst, srca0, srca1, dstb0, dstb1, rowsb0, rowsb1,
         zb, ones, cdb0, cdb1, acc, gsem0, gsem1, ssem0, ssem1) = refs[no:]
        c = jax.lax.axis_index("c")
        s = jax.lax.axis_index("s")
        w = c * 16 + s
        pltpu.sync_copy(ones_hbm, ones)

        def fill_z(i, _):
            for j in range(_HW // 16):
                zb[i, pl.ds(j * 16, 16)] = jnp.zeros((16,), jnp.float32)
            return 0
        jax.lax.fori_loop(0, _WCH, fill_z, 0)

        for t, et in enumerate(ets):
            skey, dkey = et.split('2')
            tref = tbl_refs[tbl_of[skey]]
            sref, dref, oref = src_refs[t], dst_refs[t], out_refs[t]
            nsrc = tref.shape[0] // 2
            e = sref.shape[0]
            n = oref.shape[1]
            rows = n // 16
            et_per_tile = e // 16
            zhs = [pltpu.async_copy(
                zb, acc.at[pl.ds(s * rows + j * _WCH, _WCH)], gsem0)
                for j in range(rows // _WCH)]
            for h in zhs:
                h.wait()
            plsc.subcore_barrier()

            base = s * et_per_tile
            nch = et_per_tile // _CHUNK
            pltpu.sync_copy(sref.at[pl.ds(base, et_per_tile)],
                            fsrc.at[pl.ds(0, et_per_tile)])
            pltpu.sync_copy(dref.at[pl.ds(base, et_per_tile)],
                            fdst.at[pl.ds(0, et_per_tile)])

            def prep(off, sa, db, rb, sem):
                for j in range(_CHUNK // 16):
                    sa[pl.ds(j * 16, 16)] = (
                        fsrc[pl.ds(off + j * 16, 16)] + c * nsrc)
                    db[0, pl.ds(j * 16, 16)] = fdst[pl.ds(off + j * 16, 16)]
                pltpu.async_copy(tref.at[sa], rb, sem)

            def fin(sa, db, rb, gsem, ssem):
                pltpu.make_async_copy(tref.at[sa], rb, gsem).wait()
                pltpu.async_copy(rb, acc.at[db.at[0]], ssem, add=True)

            def drain(db, rb, ssem):
                pltpu.make_async_copy(rb, acc.at[db.at[0]], ssem).wait()

            prep(0, srca0, dstb0, rowsb0, gsem0)

            def body(i2, _):
                @pl.when(i2 > 0)
                def _drainb():
                    drain(dstb1, rowsb1, ssem1)
                prep((2 * i2 + 1) * _CHUNK,
                     srca1, dstb1, rowsb1, gsem1)
                fin(srca0, dstb0, rowsb0, gsem0, ssem0)

                @pl.when(i2 + 1 < nch // 2)
                def _nexta():
                    drain(dstb0, rowsb0, ssem0)
                    prep((2 * i2 + 2) * _CHUNK,
                         srca0, dstb0, rowsb0, gsem0)
                fin(srca1, dstb1, rowsb1, gsem1, ssem1)
                return 0
            jax.lax.fori_loop(0, nch // 2, body, 0)
            drain(dstb0, rowsb0, ssem0)
            drain(dstb1, rowsb1, ssem1)
            plsc.subcore_barrier()
            whs = [pltpu.async_copy(
                acc.at[pl.ds(s * rows + j * _WCH, _WCH)],
                oref.at[c, pl.ds(s * rows + j * _WCH, _WCH)], gsem0)
                for j in range(rows // _WCH)]
            for h in whs:
                h.wait()
            plsc.subcore_barrier()

            if not with_counts:
                continue
            cref = cnt_refs[t]
            czhs = [pltpu.async_copy(
                zb, acc.at[pl.ds(s * rows + j * _WCH, _WCH)], gsem0)
                for j in range(rows // _WCH)]
            for h in czhs:
                h.wait()
            plsc.subcore_barrier()
            ew = e // 32

            ncch = ew // _CCH
            pltpu.sync_copy(dref.at[pl.ds(w * ew, ew)],
                            fdst.at[pl.ds(0, ew)])

            def cdrain(db, ssem):
                pltpu.make_async_copy(ones, acc.at[db.at[0]], ssem).wait()

            def cprep(off, db):
                for j in range(_CCH // 16):
                    db[0, pl.ds(j * 16, 16)] = fdst[pl.ds(off + j * 16, 16)]

            cprep(0, cdb0)

            def cbody(i2, _):
                @pl.when(i2 > 0)
                def _cd1():
                    cdrain(cdb1, ssem1)
                cprep((2 * i2 + 1) * _CCH, cdb1)
                pltpu.async_copy(ones, acc.at[cdb0.at[0]], ssem0, add=True)
                pltpu.async_copy(ones, acc.at[cdb1.at[0]], ssem1, add=True)

                @pl.when(i2 + 1 < ncch // 2)
                def _cnext():
                    cdrain(cdb0, ssem0)
                    cprep((2 * i2 + 2) * _CCH, cdb0)
                return 0
            jax.lax.fori_loop(0, ncch // 2, cbody, 0)
            cdrain(cdb0, ssem0)
            cdrain(cdb1, ssem1)
            plsc.subcore_barrier()
            cwhs = [pltpu.async_copy(
                acc.at[pl.ds(s * rows + j * _WCH, _WCH)],
                cref.at[c, pl.ds(s * rows + j * _WCH, _WCH)], gsem0)
                for j in range(rows // _WCH)]
            for h in cwhs:
                h.wait()
            plsc.subcore_barrier()

    return k(*tbls, *srcs, *dsts, ones_in)


def _sc_row_gather(tbl, idx):
    q = idx.shape[0]
    mesh = plsc.VectorSubcoreMesh(core_axis_name="c", subcore_axis_name="s")
    qw = q // 32

    @functools.partial(
        pl.kernel, mesh=mesh,
        out_type=jax.ShapeDtypeStruct((q, 256), jnp.float32),
        scratch_types=[
            pltpu.VMEM((_CHUNK,), jnp.int32),
            pltpu.VMEM((_CHUNK, 256), jnp.float32),
            pltpu.SemaphoreType.DMA,
        ],
    )
    def k(tbl_ref, idx_ref, out_ref, idxb, rowsb, sem):
        c = jax.lax.axis_index("c")
        s = jax.lax.axis_index("s")
        w = c * 16 + s

        def body(i, _):
            off = w * qw + i * _CHUNK
            pltpu.sync_copy(idx_ref.at[pl.ds(off, _CHUNK)], idxb)
            pltpu.async_copy(tbl_ref.at[idxb], rowsb, sem).wait()
            pltpu.sync_copy(rowsb, out_ref.at[pl.ds(off, _CHUNK)])
            return 0
        jax.lax.fori_loop(0, qw // _CHUNK, body, 0)

    return k(tbl, idx)


def kernel(x_global, x_lesion, x_cause, global_txt, text_mask, ei_g2l, ei_l2g,
           ei_l2l, ei_l2c, ei_c2l, ei_g2c, ei_c2g, cause_batch, params):
    B = x_global.shape[0]
    NL = x_lesion.shape[0]
    NC = x_cause.shape[0]
    eis = {'g2l': ei_g2l, 'l2g': ei_l2g, 'l2l': ei_l2l, 'l2c': ei_l2c,
           'c2l': ei_c2l, 'g2c': ei_g2c, 'c2g': ei_c2g}
    eis = {k: v.astype(jnp.int32) for k, v in eis.items()}
    dst_n = {'g': B, 'l': NL, 'c': NC}

    et_order = list(eis.keys())

    hg2 = _fuse_proj(x_global, global_txt, text_mask, params['gate'],
                     params['fuse'], params['proj_global'], bm=256)
    hl2 = _proj(x_lesion, params['proj_lesion'], bm=512)
    hc2 = _proj(x_cause, params['proj_cause'], bm=512)

    h2 = {'g': hg2, 'l': hl2, 'c': hc2}
    ones_in = jnp.ones((_CCH, _HW), jnp.float32)
    for bi, bp in enumerate(params['blocks']):
        h2f = {k: v.reshape(2 * v.shape[1], _HW) for k, v in h2.items()}
        out_list = _sc_block_sums(h2f, et_order,
                                  [eis[e][0] for e in et_order],
                                  [eis[e][1] for e in et_order],
                                  ones_in, dst_n, with_counts=(bi == 0))
        sums = dict(zip(et_order, out_list[:len(et_order)]))
        if bi == 0:
            cnts = {e: a[:, :, :8]
                    for e, a in zip(et_order, out_list[len(et_order):])}
        last = bi == len(params['blocks']) - 1
        new = {}
        for dst, ets, bmv in (('l', ('g2l', 'l2l', 'c2l'), 512),
                              ('g', ('l2g', 'c2g'), 256),
                              ('c', ('l2c', 'g2c'), 512)):
            new[dst] = _combine(
                h2[dst], [sums[e] for e in ets], [cnts[e] for e in ets],
                [bp[e]['Wl'] for e in ets], [bp[e]['Wr'] for e in ets],
                [bp[e]['bl'] for e in ets],
                bp['n' + dst + '_g'], bp['n' + dst + '_b'],
                bm=bmv, halves_out=not last)
        h2 = new

    hg, hl, hc = h2['g'], h2['l'], h2['c']
    ctx = _sc_row_gather(hg, cause_batch.astype(jnp.int32))
    scores = _head(hc, ctx, params['head'], bm=512)
    return scores, hc, hg, hl
